# baseline (speedup 1.0000x reference)
import functools

import jax
import jax.numpy as jnp
from jax import lax
from jax.experimental import pallas as pl
from jax.experimental.pallas import tpu as pltpu
from jax.experimental.pallas import tpu_sc as plsc

T_, B_, N_, E_ = 8, 4, 2048, 32768
RNN, NH, NO, EH, EO, OUT, DIN = 64, 64, 64, 64, 32, 6, 6
NUM_PASSING = 2

NC, NS = 2, 16
NW = NC * NS
RE = B_ * E_
RN = B_ * N_
RPW = RE // NW
CH = 128
NCH = RPW // CH

NBUF = 4


def _sc_gather_body(ps_h, pr_h, si_h, ri_h, out_h, isv, irv, bA, bB,
                    semi, semg, semw):
    wid = lax.axis_index("s") * NC + lax.axis_index("c")
    base = wid * RPW

    def group(g, carry):
        di, dg = [], []
        for j in range(NBUF):
            off = base + (g * NBUF + j) * CH
            di.append((
                pltpu.async_copy(si_h.at[pl.ds(off, CH)], isv.at[j], semi.at[j]),
                pltpu.async_copy(ri_h.at[pl.ds(off, CH)], irv.at[j], semi.at[j]),
            ))
        for j in range(NBUF):
            di[j][0].wait()
            di[j][1].wait()
            dg.append((
                pltpu.async_copy(ps_h.at[isv.at[j]], bA.at[j], semg.at[j]),
                pltpu.async_copy(pr_h.at[irv.at[j]], bB.at[j], semg.at[j]),
            ))
        dw = []
        for j in range(NBUF):
            off = base + (g * NBUF + j) * CH
            dg[j][0].wait()
            dg[j][1].wait()

            def addrow(r, c2, j=j):
                for k in range(EH // 32):
                    sl = pl.ds(k * 32, 32)
                    bA[j, r, sl] = bA[j, r, sl] + bB[j, r, sl]
                return c2

            lax.fori_loop(0, CH, addrow, 0, unroll=4)
            dw.append(pltpu.async_copy(bA.at[j], out_h.at[pl.ds(off, CH)],
                                       semw.at[j]))
        for j in range(NBUF):
            dw[j].wait()
        return carry

    lax.fori_loop(0, NCH // NBUF, group, 0)


@functools.cache
def _sc_gather_fn():
    return pl.kernel(
        _sc_gather_body,
        out_type=jax.ShapeDtypeStruct((RE, EH), jnp.bfloat16),
        mesh=plsc.VectorSubcoreMesh(core_axis_name="c", subcore_axis_name="s",
                                    num_cores=NC, num_subcores=NS),
        compiler_params=pltpu.CompilerParams(use_tc_tiling_on_sc=False),
        scratch_types=[
            pltpu.VMEM((NBUF, CH), jnp.int32),
            pltpu.VMEM((NBUF, CH), jnp.int32),
            pltpu.VMEM((NBUF, CH, EH), jnp.bfloat16),
            pltpu.VMEM((NBUF, CH, EH), jnp.bfloat16),
            pltpu.SemaphoreType.DMA((NBUF,)),
            pltpu.SemaphoreType.DMA((NBUF,)),
            pltpu.SemaphoreType.DMA((NBUF,)),
        ],
    )


def _sc_gather(ps, pr, sidx, ridx):
    return _sc_gather_fn()(ps, pr, sidx, ridx)


def _sc_scatter_body(ed_h, ri_h, out_h, idxv, ebuf, acc, semi, seme, sems):
    cid = lax.axis_index("c")
    sid = lax.axis_index("s")
    wid = sid * NC + cid
    base = wid * RPW

    def zrow(r, c2):
        for k in range(EO // 16):
            ebuf[0, r, pl.ds(k * 16, 16)] = jnp.zeros((16,), jnp.float32)
        return c2

    lax.fori_loop(0, CH, zrow, 0, unroll=2)
    rows_per_tile = RN // NS
    for j in range(rows_per_tile // CH):
        pltpu.sync_copy(ebuf.at[0],
                        acc.at[pl.ds(sid * rows_per_tile + j * CH, CH)])
    plsc.subcore_barrier()

    def group(g, carry):
        dl = []
        for j in range(NBUF):
            off = base + (g * NBUF + j) * CH
            dl.append((
                pltpu.async_copy(ri_h.at[pl.ds(off, CH)], idxv.at[j], semi.at[j]),
                pltpu.async_copy(ed_h.at[pl.ds(off, CH)], ebuf.at[j], seme.at[j]),
            ))
        ds_ = []
        for j in range(NBUF):
            dl[j][0].wait()
            dl[j][1].wait()
            ds_.append(pltpu.async_copy(ebuf.at[j], acc.at[idxv.at[j]],
                                        sems.at[j], add=True))
        for j in range(NBUF):
            ds_[j].wait()
        return carry

    lax.fori_loop(0, NCH // NBUF, group, 0)
    plsc.subcore_barrier()
    pltpu.sync_copy(acc.at[pl.ds(sid * rows_per_tile, rows_per_tile)],
                    out_h.at[pl.ds(cid * RN + sid * rows_per_tile, rows_per_tile)])


@functools.cache
def _sc_scatter_fn():
    return pl.kernel(
        _sc_scatter_body,
        out_type=jax.ShapeDtypeStruct((NC * RN, EO), jnp.float32),
        mesh=plsc.VectorSubcoreMesh(core_axis_name="c", subcore_axis_name="s",
                                    num_cores=NC, num_subcores=NS),
        compiler_params=pltpu.CompilerParams(use_tc_tiling_on_sc=False),
        scratch_types=[
            pltpu.VMEM((NBUF, CH), jnp.int32),
            pltpu.VMEM((NBUF, CH, EO), jnp.float32),
            pltpu.VMEM_SHARED((RN, EO), jnp.float32),
            pltpu.SemaphoreType.DMA((NBUF,)),
            pltpu.SemaphoreType.DMA((NBUF,)),
            pltpu.SemaphoreType.DMA((NBUF,)),
        ],
    )


def _sc_scatter(edges, ridx):
    return _sc_scatter_fn()(edges, ridx)


_BR_E = 4096
_BR_N = 1024


def _edge_tc_body(gs_ref, ew_ref, w1e_ref, b1_ref, w2_ref, b2_ref, out_ref):
    ep = jnp.dot(ew_ref[...], w1e_ref[...], preferred_element_type=jnp.float32)
    h = jnp.maximum(gs_ref[...].astype(jnp.float32) + ep + b1_ref[...], 0.0)
    out_ref[...] = jnp.dot(h, w2_ref[...],
                           preferred_element_type=jnp.float32) + b2_ref[...]


def _edge_tc(gsum, edges, w1e, b1, w2, b2):
    grid = (RE // _BR_E,)
    return pl.pallas_call(
        _edge_tc_body,
        grid=grid,
        in_specs=[
            pl.BlockSpec((_BR_E, EH), lambda i: (i, 0)),
            pl.BlockSpec((_BR_E, EO), lambda i: (i, 0)),
            pl.BlockSpec((EO, EH), lambda i: (0, 0)),
            pl.BlockSpec((1, EH), lambda i: (0, 0)),
            pl.BlockSpec((EH, EO), lambda i: (0, 0)),
            pl.BlockSpec((1, EO), lambda i: (0, 0)),
        ],
        out_specs=pl.BlockSpec((_BR_E, EO), lambda i: (i, 0)),
        out_shape=jax.ShapeDtypeStruct((RE, EO), jnp.float32),
    )(gsum, edges, w1e, b1, w2, b2)


def _proj_tc_body(x_ref, ws_ref, wr_ref, ps_ref, pr_ref):
    x = x_ref[...]
    ps_ref[...] = jnp.dot(x, ws_ref[...],
                          preferred_element_type=jnp.float32).astype(jnp.bfloat16)
    pr_ref[...] = jnp.dot(x, wr_ref[...],
                          preferred_element_type=jnp.float32).astype(jnp.bfloat16)


def _proj_tc(x, ws, wr):
    grid = (RN // _BR_N,)
    return pl.pallas_call(
        _proj_tc_body,
        grid=grid,
        in_specs=[
            pl.BlockSpec((_BR_N, NO), lambda i: (i, 0)),
            pl.BlockSpec((NO, EH), lambda i: (0, 0)),
            pl.BlockSpec((NO, EH), lambda i: (0, 0)),
        ],
        out_specs=[
            pl.BlockSpec((_BR_N, EH), lambda i: (i, 0)),
            pl.BlockSpec((_BR_N, EH), lambda i: (i, 0)),
        ],
        out_shape=[
            jax.ShapeDtypeStruct((RN, EH), jnp.bfloat16),
            jax.ShapeDtypeStruct((RN, EH), jnp.bfloat16),
        ],
    )(x, ws, wr)


def _node0_tc_body(nd_ref, pp_ref, w1n_ref, w1a_ref, b1_ref, w2_ref, b2_ref,
                   ws_ref, wr_ref, nd1_ref, ps_ref, pr_ref):
    agg = (pp_ref[0] + pp_ref[1]) * (1.0 / N_)
    h = jnp.dot(nd_ref[...], w1n_ref[...], preferred_element_type=jnp.float32)
    h = h + jnp.dot(agg, w1a_ref[...], preferred_element_type=jnp.float32)
    h = jnp.maximum(h + b1_ref[...], 0.0)
    nd1 = jnp.dot(h, w2_ref[...], preferred_element_type=jnp.float32) + b2_ref[...]
    nd1_ref[...] = nd1
    ps_ref[...] = jnp.dot(nd1, ws_ref[...],
                          preferred_element_type=jnp.float32).astype(jnp.bfloat16)
    pr_ref[...] = jnp.dot(nd1, wr_ref[...],
                          preferred_element_type=jnp.float32).astype(jnp.bfloat16)


def _node0_tc(nodes, parts, w1n, w1a, b1, w2, b2, ws_next, wr_next):
    grid = (RN // _BR_N,)
    return pl.pallas_call(
        _node0_tc_body,
        grid=grid,
        in_specs=[
            pl.BlockSpec((_BR_N, NO), lambda i: (i, 0)),
            pl.BlockSpec((NC, _BR_N, EO), lambda i: (0, i, 0)),
            pl.BlockSpec((NO, NH), lambda i: (0, 0)),
            pl.BlockSpec((EO, NH), lambda i: (0, 0)),
            pl.BlockSpec((1, NH), lambda i: (0, 0)),
            pl.BlockSpec((NH, NO), lambda i: (0, 0)),
            pl.BlockSpec((1, NO), lambda i: (0, 0)),
            pl.BlockSpec((NO, EH), lambda i: (0, 0)),
            pl.BlockSpec((NO, EH), lambda i: (0, 0)),
        ],
        out_specs=[
            pl.BlockSpec((_BR_N, NO), lambda i: (i, 0)),
            pl.BlockSpec((_BR_N, EH), lambda i: (i, 0)),
            pl.BlockSpec((_BR_N, EH), lambda i: (i, 0)),
        ],
        out_shape=[
            jax.ShapeDtypeStruct((RN, NO), jnp.float32),
            jax.ShapeDtypeStruct((RN, EH), jnp.bfloat16),
            jax.ShapeDtypeStruct((RN, EH), jnp.bfloat16),
        ],
    )(nodes, parts, w1n, w1a, b1, w2, b2, ws_next, wr_next)


def _tail_tc_body(nd_ref, pp_ref, st_ref, cur_ref, eps_ref,
                  w1n_ref, w1a_ref, b1_ref, w2_ref, b2_ref,
                  wm1_ref, bm1_ref, wm2_ref, bm2_ref,
                  ws1_ref, bs1_ref, ws2_ref, bs2_ref,
                  wih_ref, whh_ref, bih_ref, bhh_ref,
                  wps_ref, wpr_ref,
                  mean_ref, lsd_ref, smp_ref, stn_ref, ps_ref, pr_ref):
    f32 = jnp.float32
    agg = (pp_ref[0] + pp_ref[1]) * (1.0 / N_)
    h = jnp.dot(nd_ref[...], w1n_ref[...], preferred_element_type=f32)
    h = h + jnp.dot(agg, w1a_ref[...], preferred_element_type=f32)
    h = jnp.maximum(h + b1_ref[...], 0.0)
    gnn = jnp.dot(h, w2_ref[...], preferred_element_type=f32) + b2_ref[...]

    hm = jnp.maximum(jnp.dot(gnn, wm1_ref[...], preferred_element_type=f32)
                     + bm1_ref[...], 0.0)
    mean = (jnp.dot(hm, wm2_ref[...], preferred_element_type=f32)
            + bm2_ref[...] + cur_ref[...])
    hs = jnp.maximum(jnp.dot(gnn, ws1_ref[...], preferred_element_type=f32)
                     + bs1_ref[...], 0.0)
    lsd = jnp.clip(jnp.dot(hs, ws2_ref[...], preferred_element_type=f32)
                   + bs2_ref[...], -10.0, 10.0)
    mean_ref[...] = mean
    lsd_ref[...] = lsd
    smp_ref[...] = mean + jnp.exp(lsd) * eps_ref[...]

    st = st_ref[...]
    gi = jnp.dot(mean, wih_ref[...], preferred_element_type=f32) + bih_ref[...]
    gh = jnp.dot(st, whh_ref[...], preferred_element_type=f32) + bhh_ref[...]
    ir, iz, inn = gi[:, :RNN], gi[:, RNN:2 * RNN], gi[:, 2 * RNN:]
    hr, hz, hn = gh[:, :RNN], gh[:, RNN:2 * RNN], gh[:, 2 * RNN:]
    rr = jax.nn.sigmoid(ir + hr)
    z = jax.nn.sigmoid(iz + hz)
    nn_ = jnp.tanh(inn + rr * hn)
    stn = (1.0 - z) * nn_ + z * st
    stn_ref[...] = stn
    ps_ref[...] = jnp.dot(stn, wps_ref[...],
                          preferred_element_type=f32).astype(jnp.bfloat16)
    pr_ref[...] = jnp.dot(stn, wpr_ref[...],
                          preferred_element_type=f32).astype(jnp.bfloat16)


def _tail_tc(nodes, parts, state, cur, eps, P):
    grid = (RN // _BR_N,)
    row = lambda n2: pl.BlockSpec((_BR_N, n2), lambda i: (i, 0))
    full = lambda a, b: pl.BlockSpec((a, b), lambda i: (0, 0))
    return pl.pallas_call(
        _tail_tc_body,
        grid=grid,
        in_specs=[
            row(NO),
            pl.BlockSpec((NC, _BR_N, EO), lambda i: (0, i, 0)),
            row(RNN), row(OUT), row(OUT),
            full(NO, NH), full(EO, NH), full(1, NH), full(NH, NO), full(1, NO),
            full(NO, NO // 2), full(1, NO // 2), full(NO // 2, OUT), full(1, OUT),
            full(NO, NO // 2), full(1, NO // 2), full(NO // 2, OUT), full(1, OUT),
            full(DIN, 3 * RNN), full(RNN, 3 * RNN), full(1, 3 * RNN), full(1, 3 * RNN),
            full(NO, EH), full(NO, EH),
        ],
        out_specs=[row(OUT), row(OUT), row(OUT), row(RNN), row(EH), row(EH)],
        out_shape=[
            jax.ShapeDtypeStruct((RN, OUT), jnp.float32),
            jax.ShapeDtypeStruct((RN, OUT), jnp.float32),
            jax.ShapeDtypeStruct((RN, OUT), jnp.float32),
            jax.ShapeDtypeStruct((RN, RNN), jnp.float32),
            jax.ShapeDtypeStruct((RN, EH), jnp.bfloat16),
            jax.ShapeDtypeStruct((RN, EH), jnp.bfloat16),
        ],
    )(nodes, parts, state, cur, eps, *P)


def kernel(input, state, edge_weight, rec_idx, send_idx,
           W_ih, W_hh, b_ih, b_hh,
           We1_0, be1_0, We2_0, be2_0, Wn1_0, bn1_0, Wn2_0, bn2_0,
           We1_1, be1_1, We2_1, be2_1, Wn1_1, bn1_1, Wn2_1, bn2_1,
           Wm1, bm1, Wm2, bm2, Ws1, bs1, Ws2, bs2):
    f32 = jnp.float32
    st = state.reshape(RN, RNN)
    edges = edge_weight.reshape(RE, EO)
    cur = input[0].reshape(RN, DIN)

    offs = (jnp.arange(B_, dtype=jnp.int32) * N_)[:, None]
    sidx = (send_idx.astype(jnp.int32)[None, :] + offs).reshape(RE)
    ridx = (rec_idx.astype(jnp.int32)[None, :] + offs).reshape(RE)

    r1 = lambda v: v.reshape(1, -1)
    Wp = [
        dict(w1s=We1_0[:NO], w1r=We1_0[NO:2 * NO], w1e=We1_0[2 * NO:],
             b1=r1(be1_0), w2=We2_0, b2=r1(be2_0),
             wn1n=Wn1_0[:NO], wn1a=Wn1_0[NO:], bn1=r1(bn1_0),
             wn2=Wn2_0, bn2=r1(bn2_0)),
        dict(w1s=We1_1[:NO], w1r=We1_1[NO:2 * NO], w1e=We1_1[2 * NO:],
             b1=r1(be1_1), w2=We2_1, b2=r1(be2_1),
             wn1n=Wn1_1[:NO], wn1a=Wn1_1[NO:], bn1=r1(bn1_1),
             wn2=Wn2_1, bn2=r1(bn2_1)),
    ]
    tailP = (Wp[1]['wn1n'], Wp[1]['wn1a'], Wp[1]['bn1'], Wp[1]['wn2'],
             Wp[1]['bn2'],
             Wm1, r1(bm1), Wm2, r1(bm2), Ws1, r1(bs1), Ws2, r1(bs2),
             W_ih, W_hh, r1(b_ih), r1(b_hh),
             Wp[0]['w1s'], Wp[0]['w1r'])

    nkey = jax.random.key(42)
    eps_all = [jax.random.normal(jax.random.fold_in(nkey, i), (B_, N_, OUT),
                                 f32).reshape(RN, OUT) for i in range(T_)]

    ps0, pr0 = _proj_tc(st, Wp[0]['w1s'], Wp[0]['w1r'])

    means, lsds, smps, ews = [], [], [], []
    for i in range(T_):
        ews.append(edges.reshape(1, B_, E_ * EO))
        gsum = _sc_gather(ps0, pr0, sidx, ridx)
        edges = _edge_tc(gsum, edges, Wp[0]['w1e'], Wp[0]['b1'],
                         Wp[0]['w2'], Wp[0]['b2'])
        parts = _sc_scatter(edges, ridx).reshape(NC, RN, EO)
        nd1, ps1, pr1 = _node0_tc(st, parts, Wp[0]['wn1n'], Wp[0]['wn1a'],
                                  Wp[0]['bn1'], Wp[0]['wn2'], Wp[0]['bn2'],
                                  Wp[1]['w1s'], Wp[1]['w1r'])
        gsum = _sc_gather(ps1, pr1, sidx, ridx)
        edges = _edge_tc(gsum, edges, Wp[1]['w1e'], Wp[1]['b1'],
                         Wp[1]['w2'], Wp[1]['b2'])
        parts = _sc_scatter(edges, ridx).reshape(NC, RN, EO)
        mean, lsd, smp, st, ps0, pr0 = _tail_tc(nd1, parts, st, cur,
                                                eps_all[i], tailP)
        cur = mean
        means.append(mean.reshape(1, B_, N_ * OUT))
        lsds.append(lsd.reshape(1, B_, N_ * OUT))
        smps.append(smp.reshape(1, B_, N_ * OUT))

    return (jnp.concatenate(means, 0), jnp.concatenate(lsds, 0),
            jnp.concatenate(smps, 0), st.reshape(B_, N_, RNN),
            jnp.concatenate(ews, 0))


# speedup vs baseline: 1.3354x; 1.3354x over previous
import functools

import jax
import jax.numpy as jnp
from jax import lax
from jax.experimental import pallas as pl
from jax.experimental.pallas import tpu as pltpu
from jax.experimental.pallas import tpu_sc as plsc

T_, B_, N_, E_ = 8, 4, 2048, 32768
RNN, NH, NO, EH, EO, OUT, DIN = 64, 64, 64, 64, 32, 6, 6

NC, NS = 2, 16
NW = NC * NS
GW = B_ * EH
SW = B_ * EO
EPW = E_ // NW
CH = 128
NCH = EPW // CH
NBUF_G = 2
NBUF_S = 4

BF = jnp.bfloat16


def _sc_gather_body(ps_h, pr_h, si_h, ri_h, out_h, isv, irv, bA, bB,
                    semi, semg, semw):
    wid = lax.axis_index("s") * NC + lax.axis_index("c")
    base = wid * EPW

    def group(g, carry):
        di, dg = [], []
        for j in range(NBUF_G):
            off = base + (g * NBUF_G + j) * CH
            di.append((
                pltpu.async_copy(si_h.at[pl.ds(off, CH)], isv.at[j], semi.at[j]),
                pltpu.async_copy(ri_h.at[pl.ds(off, CH)], irv.at[j], semi.at[j]),
            ))
        for j in range(NBUF_G):
            di[j][0].wait()
            di[j][1].wait()
            dg.append((
                pltpu.async_copy(ps_h.at[isv.at[j]], bA.at[j], semg.at[j]),
                pltpu.async_copy(pr_h.at[irv.at[j]], bB.at[j], semg.at[j]),
            ))
        dw = []
        for j in range(NBUF_G):
            off = base + (g * NBUF_G + j) * CH
            dg[j][0].wait()
            dg[j][1].wait()

            def addrow(r, c2, j=j):
                for k in range(GW // 32):
                    sl = pl.ds(k * 32, 32)
                    bA[j, r, sl] = bA[j, r, sl] + bB[j, r, sl]
                return c2

            lax.fori_loop(0, CH, addrow, 0, unroll=4)
            dw.append(pltpu.async_copy(bA.at[j], out_h.at[pl.ds(off, CH)],
                                       semw.at[j]))
        for j in range(NBUF_G):
            dw[j].wait()
        return carry

    lax.fori_loop(0, NCH // NBUF_G, group, 0)


@functools.cache
def _sc_gather_fn():
    return pl.kernel(
        _sc_gather_body,
        out_type=jax.ShapeDtypeStruct((E_, GW), BF),
        mesh=plsc.VectorSubcoreMesh(core_axis_name="c", subcore_axis_name="s",
                                    num_cores=NC, num_subcores=NS),
        compiler_params=pltpu.CompilerParams(use_tc_tiling_on_sc=False),
        scratch_types=[
            pltpu.VMEM((NBUF_G, CH), jnp.int32),
            pltpu.VMEM((NBUF_G, CH), jnp.int32),
            pltpu.VMEM((NBUF_G, CH, GW), BF),
            pltpu.VMEM((NBUF_G, CH, GW), BF),
            pltpu.SemaphoreType.DMA((NBUF_G,)),
            pltpu.SemaphoreType.DMA((NBUF_G,)),
            pltpu.SemaphoreType.DMA((NBUF_G,)),
        ],
    )


def _sc_gather(ps, pr, sidx, ridx):
    return _sc_gather_fn()(ps, pr, sidx, ridx)


def _sc_scatter_body(ed_h, ri_h, out_h, idxv, ebuf, acc, semi, seme, sems):
    cid = lax.axis_index("c")
    sid = lax.axis_index("s")
    wid = sid * NC + cid
    base = wid * EPW

    def zrow(r, c2):
        for k in range(SW // 16):
            ebuf[0, r, pl.ds(k * 16, 16)] = jnp.zeros((16,), jnp.float32)
        return c2

    lax.fori_loop(0, CH, zrow, 0, unroll=2)
    rows_per_tile = N_ // NS
    pltpu.sync_copy(ebuf.at[0], acc.at[pl.ds(sid * rows_per_tile, CH)])
    plsc.subcore_barrier()

    def group(g, carry):
        dl = []
        for j in range(NBUF_S):
            off = base + (g * NBUF_S + j) * CH
            dl.append((
                pltpu.async_copy(ri_h.at[pl.ds(off, CH)], idxv.at[j], semi.at[j]),
                pltpu.async_copy(ed_h.at[pl.ds(off, CH)], ebuf.at[j], seme.at[j]),
            ))
        ds_ = []
        for j in range(NBUF_S):
            dl[j][0].wait()
            dl[j][1].wait()
            ds_.append(pltpu.async_copy(ebuf.at[j], acc.at[idxv.at[j]],
                                        sems.at[j], add=True))
        for j in range(NBUF_S):
            ds_[j].wait()
        return carry

    lax.fori_loop(0, NCH // NBUF_S, group, 0)
    plsc.subcore_barrier()
    pltpu.sync_copy(acc.at[pl.ds(sid * rows_per_tile, rows_per_tile)],
                    out_h.at[pl.ds(cid * N_ + sid * rows_per_tile, rows_per_tile)])


@functools.cache
def _sc_scatter_fn():
    return pl.kernel(
        _sc_scatter_body,
        out_type=jax.ShapeDtypeStruct((NC * N_, SW), jnp.float32),
        mesh=plsc.VectorSubcoreMesh(core_axis_name="c", subcore_axis_name="s",
                                    num_cores=NC, num_subcores=NS),
        compiler_params=pltpu.CompilerParams(use_tc_tiling_on_sc=False),
        scratch_types=[
            pltpu.VMEM((NBUF_S, CH), jnp.int32),
            pltpu.VMEM((NBUF_S, CH, SW), jnp.float32),
            pltpu.VMEM_SHARED((N_, SW), jnp.float32),
            pltpu.SemaphoreType.DMA((NBUF_S,)),
            pltpu.SemaphoreType.DMA((NBUF_S,)),
            pltpu.SemaphoreType.DMA((NBUF_S,)),
        ],
    )


def _sc_scatter(edgesT, ridx):
    return _sc_scatter_fn()(edgesT, ridx)


_BE = 2048
_BN = 512


def _edge_tc_body(gs_ref, ew_ref, w1e_ref, b1_ref, w2_ref, b2_ref,
                  out_ref, outT_ref):
    f32 = jnp.float32
    for b in range(B_):
        gs = gs_ref[:, b * EH:(b + 1) * EH].astype(f32)
        ep = jnp.dot(ew_ref[b], w1e_ref[...], preferred_element_type=f32)
        h = jnp.maximum(gs + ep + b1_ref[...], 0.0)
        en = jnp.dot(h, w2_ref[...], preferred_element_type=f32) + b2_ref[...]
        out_ref[b] = en
        outT_ref[:, b * EO:(b + 1) * EO] = en


def _edge_tc(gsum, edges, w1e, b1, w2, b2):
    grid = (E_ // _BE,)
    return pl.pallas_call(
        _edge_tc_body,
        grid=grid,
        in_specs=[
            pl.BlockSpec((_BE, GW), lambda i: (i, 0)),
            pl.BlockSpec((B_, _BE, EO), lambda i: (0, i, 0)),
            pl.BlockSpec((EO, EH), lambda i: (0, 0)),
            pl.BlockSpec((1, EH), lambda i: (0, 0)),
            pl.BlockSpec((EH, EO), lambda i: (0, 0)),
            pl.BlockSpec((1, EO), lambda i: (0, 0)),
        ],
        out_specs=[
            pl.BlockSpec((B_, _BE, EO), lambda i: (0, i, 0)),
            pl.BlockSpec((_BE, SW), lambda i: (i, 0)),
        ],
        out_shape=[
            jax.ShapeDtypeStruct((B_, E_, EO), jnp.float32),
            jax.ShapeDtypeStruct((E_, SW), jnp.float32),
        ],
    )(gsum, edges, w1e, b1, w2, b2)


def _proj_tc_body(x_ref, ws_ref, wr_ref, ps_ref, pr_ref):
    for b in range(B_):
        x = x_ref[b]
        ps_ref[:, b * EH:(b + 1) * EH] = jnp.dot(
            x, ws_ref[...], preferred_element_type=jnp.float32).astype(BF)
        pr_ref[:, b * EH:(b + 1) * EH] = jnp.dot(
            x, wr_ref[...], preferred_element_type=jnp.float32).astype(BF)


def _proj_tc(x, ws, wr):
    grid = (N_ // _BN,)
    return pl.pallas_call(
        _proj_tc_body,
        grid=grid,
        in_specs=[
            pl.BlockSpec((B_, _BN, NO), lambda i: (0, i, 0)),
            pl.BlockSpec((NO, EH), lambda i: (0, 0)),
            pl.BlockSpec((NO, EH), lambda i: (0, 0)),
        ],
        out_specs=[
            pl.BlockSpec((_BN, GW), lambda i: (i, 0)),
            pl.BlockSpec((_BN, GW), lambda i: (i, 0)),
        ],
        out_shape=[
            jax.ShapeDtypeStruct((N_, GW), BF),
            jax.ShapeDtypeStruct((N_, GW), BF),
        ],
    )(x, ws, wr)


def _node0_tc_body(nd_ref, pp_ref, w1n_ref, w1a_ref, b1_ref, w2_ref, b2_ref,
                   ws_ref, wr_ref, nd1_ref, ps_ref, pr_ref):
    f32 = jnp.float32
    for b in range(B_):
        agg = (pp_ref[0, :, b * EO:(b + 1) * EO]
               + pp_ref[1, :, b * EO:(b + 1) * EO]) * (1.0 / N_)
        h = jnp.dot(nd_ref[b], w1n_ref[...], preferred_element_type=f32)
        h = h + jnp.dot(agg, w1a_ref[...], preferred_element_type=f32)
        h = jnp.maximum(h + b1_ref[...], 0.0)
        nd1 = jnp.dot(h, w2_ref[...], preferred_element_type=f32) + b2_ref[...]
        nd1_ref[b] = nd1
        ps_ref[:, b * EH:(b + 1) * EH] = jnp.dot(
            nd1, ws_ref[...], preferred_element_type=f32).astype(BF)
        pr_ref[:, b * EH:(b + 1) * EH] = jnp.dot(
            nd1, wr_ref[...], preferred_element_type=f32).astype(BF)


def _node0_tc(nodes, parts, w1n, w1a, b1, w2, b2, ws_next, wr_next):
    grid = (N_ // _BN,)
    return pl.pallas_call(
        _node0_tc_body,
        grid=grid,
        in_specs=[
            pl.BlockSpec((B_, _BN, NO), lambda i: (0, i, 0)),
            pl.BlockSpec((NC, _BN, SW), lambda i: (0, i, 0)),
            pl.BlockSpec((NO, NH), lambda i: (0, 0)),
            pl.BlockSpec((EO, NH), lambda i: (0, 0)),
            pl.BlockSpec((1, NH), lambda i: (0, 0)),
            pl.BlockSpec((NH, NO), lambda i: (0, 0)),
            pl.BlockSpec((1, NO), lambda i: (0, 0)),
            pl.BlockSpec((NO, EH), lambda i: (0, 0)),
            pl.BlockSpec((NO, EH), lambda i: (0, 0)),
        ],
        out_specs=[
            pl.BlockSpec((B_, _BN, NO), lambda i: (0, i, 0)),
            pl.BlockSpec((_BN, GW), lambda i: (i, 0)),
            pl.BlockSpec((_BN, GW), lambda i: (i, 0)),
        ],
        out_shape=[
            jax.ShapeDtypeStruct((B_, N_, NO), jnp.float32),
            jax.ShapeDtypeStruct((N_, GW), BF),
            jax.ShapeDtypeStruct((N_, GW), BF),
        ],
    )(nodes, parts, w1n, w1a, b1, w2, b2, ws_next, wr_next)


def _tail_tc_body(nd_ref, pp_ref, st_ref, cur_ref, eps_ref,
                  w1n_ref, w1a_ref, b1_ref, w2_ref, b2_ref,
                  wm1_ref, bm1_ref, wm2_ref, bm2_ref,
                  ws1_ref, bs1_ref, ws2_ref, bs2_ref,
                  wih_ref, whh_ref, bih_ref, bhh_ref,
                  wps_ref, wpr_ref,
                  mean_ref, lsd_ref, smp_ref, stn_ref, ps_ref, pr_ref):
    f32 = jnp.float32
    for b in range(B_):
        agg = (pp_ref[0, :, b * EO:(b + 1) * EO]
               + pp_ref[1, :, b * EO:(b + 1) * EO]) * (1.0 / N_)
        h = jnp.dot(nd_ref[b], w1n_ref[...], preferred_element_type=f32)
        h = h + jnp.dot(agg, w1a_ref[...], preferred_element_type=f32)
        h = jnp.maximum(h + b1_ref[...], 0.0)
        gnn = jnp.dot(h, w2_ref[...], preferred_element_type=f32) + b2_ref[...]

        hm = jnp.maximum(jnp.dot(gnn, wm1_ref[...], preferred_element_type=f32)
                         + bm1_ref[...], 0.0)
        mean = (jnp.dot(hm, wm2_ref[...], preferred_element_type=f32)
                + bm2_ref[...] + cur_ref[b])
        hs = jnp.maximum(jnp.dot(gnn, ws1_ref[...], preferred_element_type=f32)
                         + bs1_ref[...], 0.0)
        lsd = jnp.clip(jnp.dot(hs, ws2_ref[...], preferred_element_type=f32)
                       + bs2_ref[...], -10.0, 10.0)
        mean_ref[b] = mean
        lsd_ref[b] = lsd
        smp_ref[b] = mean + jnp.exp(lsd) * eps_ref[b]

        st = st_ref[b]
        gi = jnp.dot(mean, wih_ref[...], preferred_element_type=f32) + bih_ref[...]
        gh = jnp.dot(st, whh_ref[...], preferred_element_type=f32) + bhh_ref[...]
        ir, iz, inn = gi[:, :RNN], gi[:, RNN:2 * RNN], gi[:, 2 * RNN:]
        hr, hz, hn = gh[:, :RNN], gh[:, RNN:2 * RNN], gh[:, 2 * RNN:]
        rr = jax.nn.sigmoid(ir + hr)
        z = jax.nn.sigmoid(iz + hz)
        nn_ = jnp.tanh(inn + rr * hn)
        stn = (1.0 - z) * nn_ + z * st
        stn_ref[b] = stn
        ps_ref[:, b * EH:(b + 1) * EH] = jnp.dot(
            stn, wps_ref[...], preferred_element_type=f32).astype(BF)
        pr_ref[:, b * EH:(b + 1) * EH] = jnp.dot(
            stn, wpr_ref[...], preferred_element_type=f32).astype(BF)


def _tail_tc(nodes, parts, state, cur, eps, P):
    grid = (N_ // _BN,)
    brow = lambda n2: pl.BlockSpec((B_, _BN, n2), lambda i: (0, i, 0))
    trow = lambda n2: pl.BlockSpec((_BN, n2), lambda i: (i, 0))
    full = lambda a, b: pl.BlockSpec((a, b), lambda i: (0, 0))
    return pl.pallas_call(
        _tail_tc_body,
        grid=grid,
        in_specs=[
            brow(NO),
            pl.BlockSpec((NC, _BN, SW), lambda i: (0, i, 0)),
            brow(RNN), brow(OUT), brow(OUT),
            full(NO, NH), full(EO, NH), full(1, NH), full(NH, NO), full(1, NO),
            full(NO, NO // 2), full(1, NO // 2), full(NO // 2, OUT), full(1, OUT),
            full(NO, NO // 2), full(1, NO // 2), full(NO // 2, OUT), full(1, OUT),
            full(DIN, 3 * RNN), full(RNN, 3 * RNN), full(1, 3 * RNN), full(1, 3 * RNN),
            full(NO, EH), full(NO, EH),
        ],
        out_specs=[brow(OUT), brow(OUT), brow(OUT), brow(RNN),
                   trow(GW), trow(GW)],
        out_shape=[
            jax.ShapeDtypeStruct((B_, N_, OUT), jnp.float32),
            jax.ShapeDtypeStruct((B_, N_, OUT), jnp.float32),
            jax.ShapeDtypeStruct((B_, N_, OUT), jnp.float32),
            jax.ShapeDtypeStruct((B_, N_, RNN), jnp.float32),
            jax.ShapeDtypeStruct((N_, GW), BF),
            jax.ShapeDtypeStruct((N_, GW), BF),
        ],
    )(nodes, parts, state, cur, eps, *P)


def kernel(input, state, edge_weight, rec_idx, send_idx,
           W_ih, W_hh, b_ih, b_hh,
           We1_0, be1_0, We2_0, be2_0, Wn1_0, bn1_0, Wn2_0, bn2_0,
           We1_1, be1_1, We2_1, be2_1, Wn1_1, bn1_1, Wn2_1, bn2_1,
           Wm1, bm1, Wm2, bm2, Ws1, bs1, Ws2, bs2):
    f32 = jnp.float32
    st = state
    edges = edge_weight
    cur = input[0]
    sidx = send_idx.astype(jnp.int32)
    ridx = rec_idx.astype(jnp.int32)

    r1 = lambda v: v.reshape(1, -1)
    Wp = [
        dict(w1s=We1_0[:NO], w1r=We1_0[NO:2 * NO], w1e=We1_0[2 * NO:],
             b1=r1(be1_0), w2=We2_0, b2=r1(be2_0),
             wn1n=Wn1_0[:NO], wn1a=Wn1_0[NO:], bn1=r1(bn1_0),
             wn2=Wn2_0, bn2=r1(bn2_0)),
        dict(w1s=We1_1[:NO], w1r=We1_1[NO:2 * NO], w1e=We1_1[2 * NO:],
             b1=r1(be1_1), w2=We2_1, b2=r1(be2_1),
             wn1n=Wn1_1[:NO], wn1a=Wn1_1[NO:], bn1=r1(bn1_1),
             wn2=Wn2_1, bn2=r1(bn2_1)),
    ]
    tailP = (Wp[1]['wn1n'], Wp[1]['wn1a'], Wp[1]['bn1'], Wp[1]['wn2'],
             Wp[1]['bn2'],
             Wm1, r1(bm1), Wm2, r1(bm2), Ws1, r1(bs1), Ws2, r1(bs2),
             W_ih, W_hh, r1(b_ih), r1(b_hh),
             Wp[0]['w1s'], Wp[0]['w1r'])

    nkey = jax.random.key(42)
    eps_all = [jax.random.normal(jax.random.fold_in(nkey, i), (B_, N_, OUT),
                                 f32) for i in range(T_)]

    ps0, pr0 = _proj_tc(st, Wp[0]['w1s'], Wp[0]['w1r'])

    means, lsds, smps, ews = [], [], [], []
    for i in range(T_):
        ews.append(edges.reshape(1, B_, E_ * EO))
        gsum = _sc_gather(ps0, pr0, sidx, ridx)
        edges, edgesT = _edge_tc(gsum, edges, Wp[0]['w1e'], Wp[0]['b1'],
                                 Wp[0]['w2'], Wp[0]['b2'])
        parts = _sc_scatter(edgesT, ridx).reshape(NC, N_, SW)
        nd1, ps1, pr1 = _node0_tc(st, parts, Wp[0]['wn1n'], Wp[0]['wn1a'],
                                  Wp[0]['bn1'], Wp[0]['wn2'], Wp[0]['bn2'],
                                  Wp[1]['w1s'], Wp[1]['w1r'])
        gsum = _sc_gather(ps1, pr1, sidx, ridx)
        edges, edgesT = _edge_tc(gsum, edges, Wp[1]['w1e'], Wp[1]['b1'],
                                 Wp[1]['w2'], Wp[1]['b2'])
        parts = _sc_scatter(edgesT, ridx).reshape(NC, N_, SW)
        mean, lsd, smp, st, ps0, pr0 = _tail_tc(nd1, parts, st, cur,
                                                eps_all[i], tailP)
        cur = mean
        means.append(mean.reshape(1, B_, N_ * OUT))
        lsds.append(lsd.reshape(1, B_, N_ * OUT))
        smps.append(smp.reshape(1, B_, N_ * OUT))

    return (jnp.concatenate(means, 0), jnp.concatenate(lsds, 0),
            jnp.concatenate(smps, 0), st,
            jnp.concatenate(ews, 0))


# speedup vs baseline: 1.3804x; 1.0337x over previous
import functools

import jax
import jax.numpy as jnp
from jax import lax
from jax.experimental import pallas as pl
from jax.experimental.pallas import tpu as pltpu
from jax.experimental.pallas import tpu_sc as plsc

T_, B_, N_, E_ = 8, 4, 2048, 32768
RNN, NH, NO, EH, EO, OUT, DIN = 64, 64, 64, 64, 32, 6, 6

NC, NS = 2, 16
NW = NC * NS
GW = B_ * EH
SW = B_ * EO
E2 = E_ // 2
CH = 128
NBUF_G = 2
NBUF_S = 4

BF = jnp.bfloat16


def _sc_gather_body(ps_h, pr_h, si_h, ri_h, out_h, isv, irv, bA, bB,
                    semi, semg, semw, *, epw):
    wid = lax.axis_index("s") * NC + lax.axis_index("c")
    base = wid * epw
    nch = epw // CH

    def group(g, carry):
        di, dg = [], []
        for j in range(NBUF_G):
            off = base + (g * NBUF_G + j) * CH
            di.append((
                pltpu.async_copy(si_h.at[pl.ds(off, CH)], isv.at[j], semi.at[j]),
                pltpu.async_copy(ri_h.at[pl.ds(off, CH)], irv.at[j], semi.at[j]),
            ))
        for j in range(NBUF_G):
            di[j][0].wait()
            di[j][1].wait()
            dg.append((
                pltpu.async_copy(ps_h.at[isv.at[j]], bA.at[j], semg.at[j]),
                pltpu.async_copy(pr_h.at[irv.at[j]], bB.at[j], semg.at[j]),
            ))
        dw = []
        for j in range(NBUF_G):
            off = base + (g * NBUF_G + j) * CH
            dg[j][0].wait()
            dg[j][1].wait()

            def addrow(r, c2, j=j):
                for k in range(GW // 32):
                    sl = pl.ds(k * 32, 32)
                    bA[j, r, sl] = bA[j, r, sl] + bB[j, r, sl]
                return c2

            lax.fori_loop(0, CH, addrow, 0, unroll=4)
            dw.append(pltpu.async_copy(bA.at[j], out_h.at[pl.ds(off, CH)],
                                       semw.at[j]))
        for j in range(NBUF_G):
            dw[j].wait()
        return carry

    lax.fori_loop(0, nch // NBUF_G, group, 0)


@functools.cache
def _sc_gather_fn(ne):
    body = functools.partial(_sc_gather_body, epw=ne // NW)
    return pl.kernel(
        body,
        out_type=jax.ShapeDtypeStruct((ne, GW), BF),
        mesh=plsc.VectorSubcoreMesh(core_axis_name="c", subcore_axis_name="s",
                                    num_cores=NC, num_subcores=NS),
        compiler_params=pltpu.CompilerParams(use_tc_tiling_on_sc=False),
        scratch_types=[
            pltpu.VMEM((NBUF_G, CH), jnp.int32),
            pltpu.VMEM((NBUF_G, CH), jnp.int32),
            pltpu.VMEM((NBUF_G, CH, GW), BF),
            pltpu.VMEM((NBUF_G, CH, GW), BF),
            pltpu.SemaphoreType.DMA((NBUF_G,)),
            pltpu.SemaphoreType.DMA((NBUF_G,)),
            pltpu.SemaphoreType.DMA((NBUF_G,)),
        ],
    )


def _sc_gather(ps, pr, sidx, ridx):
    return _sc_gather_fn(sidx.shape[0])(ps, pr, sidx, ridx)


def _sc_scatter_body(ed_h, ri_h, out_h, idxv, ebuf, acc, semi, seme, sems,
                     *, epw):
    cid = lax.axis_index("c")
    sid = lax.axis_index("s")
    wid = sid * NC + cid
    base = wid * epw
    nch = epw // CH

    def zrow(r, c2):
        for k in range(SW // 16):
            ebuf[0, r, pl.ds(k * 16, 16)] = jnp.zeros((16,), jnp.float32)
        return c2

    lax.fori_loop(0, CH, zrow, 0, unroll=2)
    rows_per_tile = N_ // NS
    pltpu.sync_copy(ebuf.at[0], acc.at[pl.ds(sid * rows_per_tile, CH)])
    plsc.subcore_barrier()

    def group(g, carry):
        dl = []
        for j in range(NBUF_S):
            off = base + (g * NBUF_S + j) * CH
            dl.append((
                pltpu.async_copy(ri_h.at[pl.ds(off, CH)], idxv.at[j], semi.at[j]),
                pltpu.async_copy(ed_h.at[pl.ds(off, CH)], ebuf.at[j], seme.at[j]),
            ))
        ds_ = []
        for j in range(NBUF_S):
            dl[j][0].wait()
            dl[j][1].wait()
            ds_.append(pltpu.async_copy(ebuf.at[j], acc.at[idxv.at[j]],
                                        sems.at[j], add=True))
        for j in range(NBUF_S):
            ds_[j].wait()
        return carry

    lax.fori_loop(0, nch // NBUF_S, group, 0)
    plsc.subcore_barrier()
    pltpu.sync_copy(acc.at[pl.ds(sid * rows_per_tile, rows_per_tile)],
                    out_h.at[pl.ds(cid * N_ + sid * rows_per_tile, rows_per_tile)])


@functools.cache
def _sc_scatter_fn(ne):
    body = functools.partial(_sc_scatter_body, epw=ne // NW)
    return pl.kernel(
        body,
        out_type=jax.ShapeDtypeStruct((NC * N_, SW), jnp.float32),
        mesh=plsc.VectorSubcoreMesh(core_axis_name="c", subcore_axis_name="s",
                                    num_cores=NC, num_subcores=NS),
        compiler_params=pltpu.CompilerParams(use_tc_tiling_on_sc=False),
        scratch_types=[
            pltpu.VMEM((NBUF_S, CH), jnp.int32),
            pltpu.VMEM((NBUF_S, CH, SW), jnp.float32),
            pltpu.VMEM_SHARED((N_, SW), jnp.float32),
            pltpu.SemaphoreType.DMA((NBUF_S,)),
            pltpu.SemaphoreType.DMA((NBUF_S,)),
            pltpu.SemaphoreType.DMA((NBUF_S,)),
        ],
    )


def _sc_scatter(edgesT, ridx):
    return _sc_scatter_fn(ridx.shape[0])(edgesT, ridx)


_BE = 2048
_BN = 512


def _edge_tc_body(gs_ref, ew_ref, w1e_ref, b1_ref, w2_ref, b2_ref,
                  out_ref, outT_ref):
    f32 = jnp.float32
    for b in range(B_):
        gs = gs_ref[:, b * EH:(b + 1) * EH].astype(f32)
        ep = jnp.dot(ew_ref[b], w1e_ref[...], preferred_element_type=f32)
        h = jnp.maximum(gs + ep + b1_ref[...], 0.0)
        en = jnp.dot(h, w2_ref[...], preferred_element_type=f32) + b2_ref[...]
        out_ref[b] = en
        outT_ref[:, b * EO:(b + 1) * EO] = en


def _edge_tc(gsum, edges, w1e, b1, w2, b2):
    ne = gsum.shape[0]
    grid = (ne // _BE,)
    return pl.pallas_call(
        _edge_tc_body,
        grid=grid,
        in_specs=[
            pl.BlockSpec((_BE, GW), lambda i: (i, 0)),
            pl.BlockSpec((B_, _BE, EO), lambda i: (0, i, 0)),
            pl.BlockSpec((EO, EH), lambda i: (0, 0)),
            pl.BlockSpec((1, EH), lambda i: (0, 0)),
            pl.BlockSpec((EH, EO), lambda i: (0, 0)),
            pl.BlockSpec((1, EO), lambda i: (0, 0)),
        ],
        out_specs=[
            pl.BlockSpec((B_, _BE, EO), lambda i: (0, i, 0)),
            pl.BlockSpec((_BE, SW), lambda i: (i, 0)),
        ],
        out_shape=[
            jax.ShapeDtypeStruct((B_, ne, EO), jnp.float32),
            jax.ShapeDtypeStruct((ne, SW), jnp.float32),
        ],
    )(gsum, edges, w1e, b1, w2, b2)


def _proj_tc_body(x_ref, ws_ref, wr_ref, ps_ref, pr_ref):
    for b in range(B_):
        x = x_ref[b]
        ps_ref[:, b * EH:(b + 1) * EH] = jnp.dot(
            x, ws_ref[...], preferred_element_type=jnp.float32).astype(BF)
        pr_ref[:, b * EH:(b + 1) * EH] = jnp.dot(
            x, wr_ref[...], preferred_element_type=jnp.float32).astype(BF)


def _proj_tc(x, ws, wr):
    grid = (N_ // _BN,)
    return pl.pallas_call(
        _proj_tc_body,
        grid=grid,
        in_specs=[
            pl.BlockSpec((B_, _BN, NO), lambda i: (0, i, 0)),
            pl.BlockSpec((NO, EH), lambda i: (0, 0)),
            pl.BlockSpec((NO, EH), lambda i: (0, 0)),
        ],
        out_specs=[
            pl.BlockSpec((_BN, GW), lambda i: (i, 0)),
            pl.BlockSpec((_BN, GW), lambda i: (i, 0)),
        ],
        out_shape=[
            jax.ShapeDtypeStruct((N_, GW), BF),
            jax.ShapeDtypeStruct((N_, GW), BF),
        ],
    )(x, ws, wr)


def _node0_tc_body(nd_ref, pp_ref, pq_ref, w1n_ref, w1a_ref, b1_ref,
                   w2_ref, b2_ref, ws_ref, wr_ref, nd1_ref, ps_ref, pr_ref):
    f32 = jnp.float32
    for b in range(B_):
        sl = slice(b * EO, (b + 1) * EO)
        agg = (pp_ref[0, :, sl] + pp_ref[1, :, sl]
               + pq_ref[0, :, sl] + pq_ref[1, :, sl]) * (1.0 / N_)
        h = jnp.dot(nd_ref[b], w1n_ref[...], preferred_element_type=f32)
        h = h + jnp.dot(agg, w1a_ref[...], preferred_element_type=f32)
        h = jnp.maximum(h + b1_ref[...], 0.0)
        nd1 = jnp.dot(h, w2_ref[...], preferred_element_type=f32) + b2_ref[...]
        nd1_ref[b] = nd1
        ps_ref[:, b * EH:(b + 1) * EH] = jnp.dot(
            nd1, ws_ref[...], preferred_element_type=f32).astype(BF)
        pr_ref[:, b * EH:(b + 1) * EH] = jnp.dot(
            nd1, wr_ref[...], preferred_element_type=f32).astype(BF)


def _node0_tc(nodes, parts, parts2, w1n, w1a, b1, w2, b2, ws_next, wr_next):
    grid = (N_ // _BN,)
    return pl.pallas_call(
        _node0_tc_body,
        grid=grid,
        in_specs=[
            pl.BlockSpec((B_, _BN, NO), lambda i: (0, i, 0)),
            pl.BlockSpec((NC, _BN, SW), lambda i: (0, i, 0)),
            pl.BlockSpec((NC, _BN, SW), lambda i: (0, i, 0)),
            pl.BlockSpec((NO, NH), lambda i: (0, 0)),
            pl.BlockSpec((EO, NH), lambda i: (0, 0)),
            pl.BlockSpec((1, NH), lambda i: (0, 0)),
            pl.BlockSpec((NH, NO), lambda i: (0, 0)),
            pl.BlockSpec((1, NO), lambda i: (0, 0)),
            pl.BlockSpec((NO, EH), lambda i: (0, 0)),
            pl.BlockSpec((NO, EH), lambda i: (0, 0)),
        ],
        out_specs=[
            pl.BlockSpec((B_, _BN, NO), lambda i: (0, i, 0)),
            pl.BlockSpec((_BN, GW), lambda i: (i, 0)),
            pl.BlockSpec((_BN, GW), lambda i: (i, 0)),
        ],
        out_shape=[
            jax.ShapeDtypeStruct((B_, N_, NO), jnp.float32),
            jax.ShapeDtypeStruct((N_, GW), BF),
            jax.ShapeDtypeStruct((N_, GW), BF),
        ],
    )(nodes, parts, parts2, w1n, w1a, b1, w2, b2, ws_next, wr_next)


def _tail_tc_body(nd_ref, pp_ref, pq_ref, st_ref, cur_ref, eps_ref,
                  w1n_ref, w1a_ref, b1_ref, w2_ref, b2_ref,
                  wm1_ref, bm1_ref, wm2_ref, bm2_ref,
                  ws1_ref, bs1_ref, ws2_ref, bs2_ref,
                  wih_ref, whh_ref, bih_ref, bhh_ref,
                  wps_ref, wpr_ref,
                  mean_ref, lsd_ref, smp_ref, stn_ref, ps_ref, pr_ref):
    f32 = jnp.float32
    for b in range(B_):
        sl = slice(b * EO, (b + 1) * EO)
        agg = (pp_ref[0, :, sl] + pp_ref[1, :, sl]
               + pq_ref[0, :, sl] + pq_ref[1, :, sl]) * (1.0 / N_)
        h = jnp.dot(nd_ref[b], w1n_ref[...], preferred_element_type=f32)
        h = h + jnp.dot(agg, w1a_ref[...], preferred_element_type=f32)
        h = jnp.maximum(h + b1_ref[...], 0.0)
        gnn = jnp.dot(h, w2_ref[...], preferred_element_type=f32) + b2_ref[...]

        hm = jnp.maximum(jnp.dot(gnn, wm1_ref[...], preferred_element_type=f32)
                         + bm1_ref[...], 0.0)
        mean = (jnp.dot(hm, wm2_ref[...], preferred_element_type=f32)
                + bm2_ref[...] + cur_ref[b])
        hs = jnp.maximum(jnp.dot(gnn, ws1_ref[...], preferred_element_type=f32)
                         + bs1_ref[...], 0.0)
        lsd = jnp.clip(jnp.dot(hs, ws2_ref[...], preferred_element_type=f32)
                       + bs2_ref[...], -10.0, 10.0)
        mean_ref[b] = mean
        lsd_ref[b] = lsd
        smp_ref[b] = mean + jnp.exp(lsd) * eps_ref[b]

        st = st_ref[b]
        gi = jnp.dot(mean, wih_ref[...], preferred_element_type=f32) + bih_ref[...]
        gh = jnp.dot(st, whh_ref[...], preferred_element_type=f32) + bhh_ref[...]
        ir, iz, inn = gi[:, :RNN], gi[:, RNN:2 * RNN], gi[:, 2 * RNN:]
        hr, hz, hn = gh[:, :RNN], gh[:, RNN:2 * RNN], gh[:, 2 * RNN:]
        rr = jax.nn.sigmoid(ir + hr)
        z = jax.nn.sigmoid(iz + hz)
        nn_ = jnp.tanh(inn + rr * hn)
        stn = (1.0 - z) * nn_ + z * st
        stn_ref[b] = stn
        ps_ref[:, b * EH:(b + 1) * EH] = jnp.dot(
            stn, wps_ref[...], preferred_element_type=f32).astype(BF)
        pr_ref[:, b * EH:(b + 1) * EH] = jnp.dot(
            stn, wpr_ref[...], preferred_element_type=f32).astype(BF)


def _tail_tc(nodes, parts, parts2, state, cur, eps, P):
    grid = (N_ // _BN,)
    brow = lambda n2: pl.BlockSpec((B_, _BN, n2), lambda i: (0, i, 0))
    trow = lambda n2: pl.BlockSpec((_BN, n2), lambda i: (i, 0))
    full = lambda a, b: pl.BlockSpec((a, b), lambda i: (0, 0))
    return pl.pallas_call(
        _tail_tc_body,
        grid=grid,
        in_specs=[
            brow(NO),
            pl.BlockSpec((NC, _BN, SW), lambda i: (0, i, 0)),
            pl.BlockSpec((NC, _BN, SW), lambda i: (0, i, 0)),
            brow(RNN), brow(OUT), brow(OUT),
            full(NO, NH), full(EO, NH), full(1, NH), full(NH, NO), full(1, NO),
            full(NO, NO // 2), full(1, NO // 2), full(NO // 2, OUT), full(1, OUT),
            full(NO, NO // 2), full(1, NO // 2), full(NO // 2, OUT), full(1, OUT),
            full(DIN, 3 * RNN), full(RNN, 3 * RNN), full(1, 3 * RNN), full(1, 3 * RNN),
            full(NO, EH), full(NO, EH),
        ],
        out_specs=[brow(OUT), brow(OUT), brow(OUT), brow(RNN),
                   trow(GW), trow(GW)],
        out_shape=[
            jax.ShapeDtypeStruct((B_, N_, OUT), jnp.float32),
            jax.ShapeDtypeStruct((B_, N_, OUT), jnp.float32),
            jax.ShapeDtypeStruct((B_, N_, OUT), jnp.float32),
            jax.ShapeDtypeStruct((B_, N_, RNN), jnp.float32),
            jax.ShapeDtypeStruct((N_, GW), BF),
            jax.ShapeDtypeStruct((N_, GW), BF),
        ],
    )(nodes, parts, parts2, state, cur, eps, *P)


def kernel(input, state, edge_weight, rec_idx, send_idx,
           W_ih, W_hh, b_ih, b_hh,
           We1_0, be1_0, We2_0, be2_0, Wn1_0, bn1_0, Wn2_0, bn2_0,
           We1_1, be1_1, We2_1, be2_1, Wn1_1, bn1_1, Wn2_1, bn2_1,
           Wm1, bm1, Wm2, bm2, Ws1, bs1, Ws2, bs2):
    f32 = jnp.float32
    st = state
    ed0 = edge_weight[:, :E2]
    ed1 = edge_weight[:, E2:]
    cur = input[0]
    sidx = send_idx.astype(jnp.int32)
    ridx = rec_idx.astype(jnp.int32)
    si0, si1 = sidx[:E2], sidx[E2:]
    ri0, ri1 = ridx[:E2], ridx[E2:]

    r1 = lambda v: v.reshape(1, -1)
    Wp = [
        dict(w1s=We1_0[:NO], w1r=We1_0[NO:2 * NO], w1e=We1_0[2 * NO:],
             b1=r1(be1_0), w2=We2_0, b2=r1(be2_0),
             wn1n=Wn1_0[:NO], wn1a=Wn1_0[NO:], bn1=r1(bn1_0),
             wn2=Wn2_0, bn2=r1(bn2_0)),
        dict(w1s=We1_1[:NO], w1r=We1_1[NO:2 * NO], w1e=We1_1[2 * NO:],
             b1=r1(be1_1), w2=We2_1, b2=r1(be2_1),
             wn1n=Wn1_1[:NO], wn1a=Wn1_1[NO:], bn1=r1(bn1_1),
             wn2=Wn2_1, bn2=r1(bn2_1)),
    ]
    tailP = (Wp[1]['wn1n'], Wp[1]['wn1a'], Wp[1]['bn1'], Wp[1]['wn2'],
             Wp[1]['bn2'],
             Wm1, r1(bm1), Wm2, r1(bm2), Ws1, r1(bs1), Ws2, r1(bs2),
             W_ih, W_hh, r1(b_ih), r1(b_hh),
             Wp[0]['w1s'], Wp[0]['w1r'])

    nkey = jax.random.key(42)
    eps_all = [jax.random.normal(jax.random.fold_in(nkey, i), (B_, N_, OUT),
                                 f32) for i in range(T_)]

    ps0, pr0 = _proj_tc(st, Wp[0]['w1s'], Wp[0]['w1r'])

    means, lsds, smps, ews = [], [], [], []
    for i in range(T_):
        ews.append(jnp.concatenate(
            [ed0.reshape(B_, E2 * EO), ed1.reshape(B_, E2 * EO)],
            axis=1).reshape(1, B_, E_ * EO))
        for p in range(2):
            g0 = _sc_gather(ps0, pr0, si0, ri0)
            g1 = _sc_gather(ps0, pr0, si1, ri1)
            ed0, e0T = _edge_tc(g0, ed0, Wp[p]['w1e'], Wp[p]['b1'],
                                Wp[p]['w2'], Wp[p]['b2'])
            ed1, e1T = _edge_tc(g1, ed1, Wp[p]['w1e'], Wp[p]['b1'],
                                Wp[p]['w2'], Wp[p]['b2'])
            pa = _sc_scatter(e0T, ri0).reshape(NC, N_, SW)
            pb = _sc_scatter(e1T, ri1).reshape(NC, N_, SW)
            if p == 0:
                nd1, ps0, pr0 = _node0_tc(st, pa, pb, Wp[0]['wn1n'],
                                          Wp[0]['wn1a'], Wp[0]['bn1'],
                                          Wp[0]['wn2'], Wp[0]['bn2'],
                                          Wp[1]['w1s'], Wp[1]['w1r'])
            else:
                mean, lsd, smp, st, ps0, pr0 = _tail_tc(nd1, pa, pb, st, cur,
                                                        eps_all[i], tailP)
        cur = mean
        means.append(mean.reshape(1, B_, N_ * OUT))
        lsds.append(lsd.reshape(1, B_, N_ * OUT))
        smps.append(smp.reshape(1, B_, N_ * OUT))

    return (jnp.concatenate(means, 0), jnp.concatenate(lsds, 0),
            jnp.concatenate(smps, 0), st,
            jnp.concatenate(ews, 0))


# speedup vs baseline: 1.5164x; 1.0985x over previous
import functools

import jax
import jax.numpy as jnp
from jax import lax
from jax.experimental import pallas as pl
from jax.experimental.pallas import tpu as pltpu
from jax.experimental.pallas import tpu_sc as plsc

T_, B_, N_, E_ = 8, 4, 2048, 32768
RNN, NH, NO, EH, EO, OUT, DIN = 64, 64, 64, 64, 32, 6, 6

NC, NS = 2, 16
NW = NC * NS
GW = B_ * EH
SW = B_ * EO
E2 = E_ // 2
CH = 128
CHG = 64
NBUF_G = 2
NBUF_S = 4


def _sc_gather_body(ps_h, pr_h, si_h, ri_h, out_h, isv, irv, bA, bB,
                    semi, semg, semw, *, epw):
    wid = lax.axis_index("s") * NC + lax.axis_index("c")
    base = wid * epw
    nch = epw // CHG

    def group(g, carry):
        di, dg = [], []
        for j in range(NBUF_G):
            off = base + (g * NBUF_G + j) * CHG
            di.append((
                pltpu.async_copy(si_h.at[pl.ds(off, CHG)], isv.at[j], semi.at[j]),
                pltpu.async_copy(ri_h.at[pl.ds(off, CHG)], irv.at[j], semi.at[j]),
            ))
        for j in range(NBUF_G):
            di[j][0].wait()
            di[j][1].wait()
            dg.append((
                pltpu.async_copy(ps_h.at[isv.at[j]], bA.at[j], semg.at[j]),
                pltpu.async_copy(pr_h.at[irv.at[j]], bB.at[j], semg.at[j]),
            ))
        dw = []
        for j in range(NBUF_G):
            off = base + (g * NBUF_G + j) * CHG
            dg[j][0].wait()
            dg[j][1].wait()

            def addrow(r, c2, j=j):
                for k in range(GW // 16):
                    sl = pl.ds(k * 16, 16)
                    bA[j, r, sl] = bA[j, r, sl] + bB[j, r, sl]
                return c2

            lax.fori_loop(0, CHG, addrow, 0, unroll=4)
            dw.append(pltpu.async_copy(bA.at[j], out_h.at[pl.ds(off, CHG)],
                                       semw.at[j]))
        for j in range(NBUF_G):
            dw[j].wait()
        return carry

    lax.fori_loop(0, nch // NBUF_G, group, 0)


@functools.cache
def _sc_gather_fn(ne):
    body = functools.partial(_sc_gather_body, epw=ne // NW)
    return pl.kernel(
        body,
        out_type=jax.ShapeDtypeStruct((ne, GW), jnp.float32),
        mesh=plsc.VectorSubcoreMesh(core_axis_name="c", subcore_axis_name="s",
                                    num_cores=NC, num_subcores=NS),
        scratch_types=[
            pltpu.VMEM((NBUF_G, CHG), jnp.int32),
            pltpu.VMEM((NBUF_G, CHG), jnp.int32),
            pltpu.VMEM((NBUF_G, CHG, GW), jnp.float32),
            pltpu.VMEM((NBUF_G, CHG, GW), jnp.float32),
            pltpu.SemaphoreType.DMA((NBUF_G,)),
            pltpu.SemaphoreType.DMA((NBUF_G,)),
            pltpu.SemaphoreType.DMA((NBUF_G,)),
        ],
    )


def _sc_gather(ps, pr, sidx, ridx):
    return _sc_gather_fn(sidx.shape[0])(ps, pr, sidx, ridx)


def _sc_scatter_body(ed_h, ri_h, out_h, idxv, ebuf, acc, semi, seme, sems,
                     *, epw):
    cid = lax.axis_index("c")
    sid = lax.axis_index("s")
    wid = sid * NC + cid
    base = wid * epw
    nch = epw // CH

    def zrow(r, c2):
        for k in range(SW // 16):
            ebuf[0, r, pl.ds(k * 16, 16)] = jnp.zeros((16,), jnp.float32)
        return c2

    lax.fori_loop(0, CH, zrow, 0, unroll=2)
    rows_per_tile = N_ // NS
    pltpu.sync_copy(ebuf.at[0], acc.at[pl.ds(sid * rows_per_tile, CH)])
    plsc.subcore_barrier()

    def group(g, carry):
        dl = []
        for j in range(NBUF_S):
            off = base + (g * NBUF_S + j) * CH
            dl.append((
                pltpu.async_copy(ri_h.at[pl.ds(off, CH)], idxv.at[j], semi.at[j]),
                pltpu.async_copy(ed_h.at[pl.ds(off, CH)], ebuf.at[j], seme.at[j]),
            ))
        ds_ = []
        for j in range(NBUF_S):
            dl[j][0].wait()
            dl[j][1].wait()
            ds_.append(pltpu.async_copy(ebuf.at[j], acc.at[idxv.at[j]],
                                        sems.at[j], add=True))
        for j in range(NBUF_S):
            ds_[j].wait()
        return carry

    lax.fori_loop(0, nch // NBUF_S, group, 0)
    plsc.subcore_barrier()
    pltpu.sync_copy(acc.at[pl.ds(sid * rows_per_tile, rows_per_tile)],
                    out_h.at[pl.ds(cid * N_ + sid * rows_per_tile, rows_per_tile)])


@functools.cache
def _sc_scatter_fn(ne):
    body = functools.partial(_sc_scatter_body, epw=ne // NW)
    return pl.kernel(
        body,
        out_type=jax.ShapeDtypeStruct((NC * N_, SW), jnp.float32),
        mesh=plsc.VectorSubcoreMesh(core_axis_name="c", subcore_axis_name="s",
                                    num_cores=NC, num_subcores=NS),
        scratch_types=[
            pltpu.VMEM((NBUF_S, CH), jnp.int32),
            pltpu.VMEM((NBUF_S, CH, SW), jnp.float32),
            pltpu.VMEM_SHARED((N_, SW), jnp.float32),
            pltpu.SemaphoreType.DMA((NBUF_S,)),
            pltpu.SemaphoreType.DMA((NBUF_S,)),
            pltpu.SemaphoreType.DMA((NBUF_S,)),
        ],
    )


def _sc_scatter(edgesT, ridx):
    return _sc_scatter_fn(ridx.shape[0])(edgesT, ridx)


_BE = 2048
_BN = 512


def _edge_tc_body(gs_ref, ew_ref, w1e_ref, b1_ref, w2_ref, b2_ref, outT_ref):
    f32 = jnp.float32
    for b in range(B_):
        gs = gs_ref[:, b * EH:(b + 1) * EH]
        ep = jnp.dot(ew_ref[:, b * EO:(b + 1) * EO], w1e_ref[...],
                     preferred_element_type=f32)
        h = jnp.maximum(gs + ep + b1_ref[...], 0.0)
        en = jnp.dot(h, w2_ref[...], preferred_element_type=f32) + b2_ref[...]
        outT_ref[:, b * EO:(b + 1) * EO] = en


def _edge_tc(gsum, edgesT, w1e, b1, w2, b2):
    ne = gsum.shape[0]
    grid = (ne // _BE,)
    return pl.pallas_call(
        _edge_tc_body,
        grid=grid,
        in_specs=[
            pl.BlockSpec((_BE, GW), lambda i: (i, 0)),
            pl.BlockSpec((_BE, SW), lambda i: (i, 0)),
            pl.BlockSpec((EO, EH), lambda i: (0, 0)),
            pl.BlockSpec((1, EH), lambda i: (0, 0)),
            pl.BlockSpec((EH, EO), lambda i: (0, 0)),
            pl.BlockSpec((1, EO), lambda i: (0, 0)),
        ],
        out_specs=pl.BlockSpec((_BE, SW), lambda i: (i, 0)),
        out_shape=jax.ShapeDtypeStruct((ne, SW), jnp.float32),
    )(gsum, edgesT, w1e, b1, w2, b2)


def _proj_tc_body(x_ref, ws_ref, wr_ref, ps_ref, pr_ref):
    for b in range(B_):
        x = x_ref[b]
        ps_ref[:, b * EH:(b + 1) * EH] = jnp.dot(
            x, ws_ref[...], preferred_element_type=jnp.float32)
        pr_ref[:, b * EH:(b + 1) * EH] = jnp.dot(
            x, wr_ref[...], preferred_element_type=jnp.float32)


def _proj_tc(x, ws, wr):
    grid = (N_ // _BN,)
    return pl.pallas_call(
        _proj_tc_body,
        grid=grid,
        in_specs=[
            pl.BlockSpec((B_, _BN, NO), lambda i: (0, i, 0)),
            pl.BlockSpec((NO, EH), lambda i: (0, 0)),
            pl.BlockSpec((NO, EH), lambda i: (0, 0)),
        ],
        out_specs=[
            pl.BlockSpec((_BN, GW), lambda i: (i, 0)),
            pl.BlockSpec((_BN, GW), lambda i: (i, 0)),
        ],
        out_shape=[
            jax.ShapeDtypeStruct((N_, GW), jnp.float32),
            jax.ShapeDtypeStruct((N_, GW), jnp.float32),
        ],
    )(x, ws, wr)


def _node0_tc_body(nd_ref, pp_ref, pq_ref, w1n_ref, w1a_ref, b1_ref,
                   w2_ref, b2_ref, ws_ref, wr_ref, nd1_ref, ps_ref, pr_ref):
    f32 = jnp.float32
    for b in range(B_):
        sl = slice(b * EO, (b + 1) * EO)
        agg = (pp_ref[0, :, sl] + pp_ref[1, :, sl]
               + pq_ref[0, :, sl] + pq_ref[1, :, sl]) * (1.0 / N_)
        h = jnp.dot(nd_ref[b], w1n_ref[...], preferred_element_type=f32)
        h = h + jnp.dot(agg, w1a_ref[...], preferred_element_type=f32)
        h = jnp.maximum(h + b1_ref[...], 0.0)
        nd1 = jnp.dot(h, w2_ref[...], preferred_element_type=f32) + b2_ref[...]
        nd1_ref[b] = nd1
        ps_ref[:, b * EH:(b + 1) * EH] = jnp.dot(
            nd1, ws_ref[...], preferred_element_type=f32)
        pr_ref[:, b * EH:(b + 1) * EH] = jnp.dot(
            nd1, wr_ref[...], preferred_element_type=f32)


def _node0_tc(nodes, parts, parts2, w1n, w1a, b1, w2, b2, ws_next, wr_next):
    grid = (N_ // _BN,)
    return pl.pallas_call(
        _node0_tc_body,
        grid=grid,
        in_specs=[
            pl.BlockSpec((B_, _BN, NO), lambda i: (0, i, 0)),
            pl.BlockSpec((NC, _BN, SW), lambda i: (0, i, 0)),
            pl.BlockSpec((NC, _BN, SW), lambda i: (0, i, 0)),
            pl.BlockSpec((NO, NH), lambda i: (0, 0)),
            pl.BlockSpec((EO, NH), lambda i: (0, 0)),
            pl.BlockSpec((1, NH), lambda i: (0, 0)),
            pl.BlockSpec((NH, NO), lambda i: (0, 0)),
            pl.BlockSpec((1, NO), lambda i: (0, 0)),
            pl.BlockSpec((NO, EH), lambda i: (0, 0)),
            pl.BlockSpec((NO, EH), lambda i: (0, 0)),
        ],
        out_specs=[
            pl.BlockSpec((B_, _BN, NO), lambda i: (0, i, 0)),
            pl.BlockSpec((_BN, GW), lambda i: (i, 0)),
            pl.BlockSpec((_BN, GW), lambda i: (i, 0)),
        ],
        out_shape=[
            jax.ShapeDtypeStruct((B_, N_, NO), jnp.float32),
            jax.ShapeDtypeStruct((N_, GW), jnp.float32),
            jax.ShapeDtypeStruct((N_, GW), jnp.float32),
        ],
    )(nodes, parts, parts2, w1n, w1a, b1, w2, b2, ws_next, wr_next)


def _tail_tc_body(nd_ref, pp_ref, pq_ref, st_ref, cur_ref, eps_ref,
                  w1n_ref, w1a_ref, b1_ref, w2_ref, b2_ref,
                  wm1_ref, bm1_ref, wm2_ref, bm2_ref,
                  ws1_ref, bs1_ref, ws2_ref, bs2_ref,
                  wih_ref, whh_ref, bih_ref, bhh_ref,
                  wps_ref, wpr_ref,
                  mean_ref, lsd_ref, smp_ref, stn_ref, ps_ref, pr_ref):
    f32 = jnp.float32
    for b in range(B_):
        sl = slice(b * EO, (b + 1) * EO)
        agg = (pp_ref[0, :, sl] + pp_ref[1, :, sl]
               + pq_ref[0, :, sl] + pq_ref[1, :, sl]) * (1.0 / N_)
        h = jnp.dot(nd_ref[b], w1n_ref[...], preferred_element_type=f32)
        h = h + jnp.dot(agg, w1a_ref[...], preferred_element_type=f32)
        h = jnp.maximum(h + b1_ref[...], 0.0)
        gnn = jnp.dot(h, w2_ref[...], preferred_element_type=f32) + b2_ref[...]

        hm = jnp.maximum(jnp.dot(gnn, wm1_ref[...], preferred_element_type=f32)
                         + bm1_ref[...], 0.0)
        mean = (jnp.dot(hm, wm2_ref[...], preferred_element_type=f32)
                + bm2_ref[...] + cur_ref[b])
        hs = jnp.maximum(jnp.dot(gnn, ws1_ref[...], preferred_element_type=f32)
                         + bs1_ref[...], 0.0)
        lsd = jnp.clip(jnp.dot(hs, ws2_ref[...], preferred_element_type=f32)
                       + bs2_ref[...], -10.0, 10.0)
        mean_ref[b] = mean
        lsd_ref[b] = lsd
        smp_ref[b] = mean + jnp.exp(lsd) * eps_ref[b]

        st = st_ref[b]
        gi = jnp.dot(mean, wih_ref[...], preferred_element_type=f32) + bih_ref[...]
        gh = jnp.dot(st, whh_ref[...], preferred_element_type=f32) + bhh_ref[...]
        ir, iz, inn = gi[:, :RNN], gi[:, RNN:2 * RNN], gi[:, 2 * RNN:]
        hr, hz, hn = gh[:, :RNN], gh[:, RNN:2 * RNN], gh[:, 2 * RNN:]
        rr = jax.nn.sigmoid(ir + hr)
        z = jax.nn.sigmoid(iz + hz)
        nn_ = jnp.tanh(inn + rr * hn)
        stn = (1.0 - z) * nn_ + z * st
        stn_ref[b] = stn
        ps_ref[:, b * EH:(b + 1) * EH] = jnp.dot(
            stn, wps_ref[...], preferred_element_type=f32)
        pr_ref[:, b * EH:(b + 1) * EH] = jnp.dot(
            stn, wpr_ref[...], preferred_element_type=f32)


def _tail_tc(nodes, parts, parts2, state, cur, eps, P):
    grid = (N_ // _BN,)
    brow = lambda n2: pl.BlockSpec((B_, _BN, n2), lambda i: (0, i, 0))
    trow = lambda n2: pl.BlockSpec((_BN, n2), lambda i: (i, 0))
    full = lambda a, b: pl.BlockSpec((a, b), lambda i: (0, 0))
    return pl.pallas_call(
        _tail_tc_body,
        grid=grid,
        in_specs=[
            brow(NO),
            pl.BlockSpec((NC, _BN, SW), lambda i: (0, i, 0)),
            pl.BlockSpec((NC, _BN, SW), lambda i: (0, i, 0)),
            brow(RNN), brow(OUT), brow(OUT),
            full(NO, NH), full(EO, NH), full(1, NH), full(NH, NO), full(1, NO),
            full(NO, NO // 2), full(1, NO // 2), full(NO // 2, OUT), full(1, OUT),
            full(NO, NO // 2), full(1, NO // 2), full(NO // 2, OUT), full(1, OUT),
            full(DIN, 3 * RNN), full(RNN, 3 * RNN), full(1, 3 * RNN), full(1, 3 * RNN),
            full(NO, EH), full(NO, EH),
        ],
        out_specs=[brow(OUT), brow(OUT), brow(OUT), brow(RNN),
                   trow(GW), trow(GW)],
        out_shape=[
            jax.ShapeDtypeStruct((B_, N_, OUT), jnp.float32),
            jax.ShapeDtypeStruct((B_, N_, OUT), jnp.float32),
            jax.ShapeDtypeStruct((B_, N_, OUT), jnp.float32),
            jax.ShapeDtypeStruct((B_, N_, RNN), jnp.float32),
            jax.ShapeDtypeStruct((N_, GW), jnp.float32),
            jax.ShapeDtypeStruct((N_, GW), jnp.float32),
        ],
    )(nodes, parts, parts2, state, cur, eps, *P)


def kernel(input, state, edge_weight, rec_idx, send_idx,
           W_ih, W_hh, b_ih, b_hh,
           We1_0, be1_0, We2_0, be2_0, Wn1_0, bn1_0, Wn2_0, bn2_0,
           We1_1, be1_1, We2_1, be2_1, Wn1_1, bn1_1, Wn2_1, bn2_1,
           Wm1, bm1, Wm2, bm2, Ws1, bs1, Ws2, bs2):
    f32 = jnp.float32
    st = state
    edT = edge_weight.transpose(1, 0, 2).reshape(E_, SW)
    ed0, ed1 = edT[:E2], edT[E2:]
    cur = input[0]
    sidx = send_idx.astype(jnp.int32)
    ridx = rec_idx.astype(jnp.int32)
    si0, si1 = sidx[:E2], sidx[E2:]
    ri0, ri1 = ridx[:E2], ridx[E2:]

    r1 = lambda v: v.reshape(1, -1)
    Wp = [
        dict(w1s=We1_0[:NO], w1r=We1_0[NO:2 * NO], w1e=We1_0[2 * NO:],
             b1=r1(be1_0), w2=We2_0, b2=r1(be2_0),
             wn1n=Wn1_0[:NO], wn1a=Wn1_0[NO:], bn1=r1(bn1_0),
             wn2=Wn2_0, bn2=r1(bn2_0)),
        dict(w1s=We1_1[:NO], w1r=We1_1[NO:2 * NO], w1e=We1_1[2 * NO:],
             b1=r1(be1_1), w2=We2_1, b2=r1(be2_1),
             wn1n=Wn1_1[:NO], wn1a=Wn1_1[NO:], bn1=r1(bn1_1),
             wn2=Wn2_1, bn2=r1(bn2_1)),
    ]
    tailP = (Wp[1]['wn1n'], Wp[1]['wn1a'], Wp[1]['bn1'], Wp[1]['wn2'],
             Wp[1]['bn2'],
             Wm1, r1(bm1), Wm2, r1(bm2), Ws1, r1(bs1), Ws2, r1(bs2),
             W_ih, W_hh, r1(b_ih), r1(b_hh),
             Wp[0]['w1s'], Wp[0]['w1r'])

    nkey = jax.random.key(42)
    eps_all = [jax.random.normal(jax.random.fold_in(nkey, i), (B_, N_, OUT),
                                 f32) for i in range(T_)]

    ps0, pr0 = _proj_tc(st, Wp[0]['w1s'], Wp[0]['w1r'])

    means, lsds, smps, ews = [], [], [], []
    for i in range(T_):
        ews.append((ed0, ed1))
        for p in range(2):
            g0 = _sc_gather(ps0, pr0, si0, ri0)
            g1 = _sc_gather(ps0, pr0, si1, ri1)
            ed0 = _edge_tc(g0, ed0, Wp[p]['w1e'], Wp[p]['b1'],
                           Wp[p]['w2'], Wp[p]['b2'])
            ed1 = _edge_tc(g1, ed1, Wp[p]['w1e'], Wp[p]['b1'],
                           Wp[p]['w2'], Wp[p]['b2'])
            pa = _sc_scatter(ed0, ri0).reshape(NC, N_, SW)
            pb = _sc_scatter(ed1, ri1).reshape(NC, N_, SW)
            if p == 0:
                nd1, ps0, pr0 = _node0_tc(st, pa, pb, Wp[0]['wn1n'],
                                          Wp[0]['wn1a'], Wp[0]['bn1'],
                                          Wp[0]['wn2'], Wp[0]['bn2'],
                                          Wp[1]['w1s'], Wp[1]['w1r'])
            else:
                mean, lsd, smp, st, ps0, pr0 = _tail_tc(nd1, pa, pb, st, cur,
                                                        eps_all[i], tailP)
        cur = mean
        means.append(mean.reshape(1, B_, N_ * OUT))
        lsds.append(lsd.reshape(1, B_, N_ * OUT))
        smps.append(smp.reshape(1, B_, N_ * OUT))

    ewsT = jnp.stack([jnp.concatenate(h, axis=0) for h in ews])
    ews_out = ewsT.reshape(T_, E_, B_, EO).transpose(0, 2, 1, 3).reshape(
        T_, B_, E_ * EO)
    return (jnp.concatenate(means, 0), jnp.concatenate(lsds, 0),
            jnp.concatenate(smps, 0), st, ews_out)


# speedup vs baseline: 1.6197x; 1.0681x over previous
import functools

import jax
import jax.numpy as jnp
from jax import lax
from jax.experimental import pallas as pl
from jax.experimental.pallas import tpu as pltpu
from jax.experimental.pallas import tpu_sc as plsc

T_, B_, N_, E_ = 8, 4, 2048, 32768
RNN, NH, NO, EH, EO, OUT, DIN = 64, 64, 64, 64, 32, 6, 6

NC, NS = 2, 16
NW = NC * NS
GW = B_ * EH
SW = B_ * EO
E2 = E_ // 2
CH = 128
CHG = 64
NBUF_G = 2
NBUF_S = 4


def _sc_gather_body(ps_h, pr_h, si_h, ri_h, out_h, isv, irv, bA, bB,
                    semi, semg, semw, *, epw):
    wid = lax.axis_index("s") * NC + lax.axis_index("c")
    base = wid * epw
    nch = epw // CHG

    def group(g, carry):
        di, dg = [], []
        for j in range(NBUF_G):
            off = base + (g * NBUF_G + j) * CHG
            di.append((
                pltpu.async_copy(si_h.at[pl.ds(off, CHG)], isv.at[j], semi.at[j]),
                pltpu.async_copy(ri_h.at[pl.ds(off, CHG)], irv.at[j], semi.at[j]),
            ))
        for j in range(NBUF_G):
            di[j][0].wait()
            di[j][1].wait()
            dg.append((
                pltpu.async_copy(ps_h.at[isv.at[j]], bA.at[j], semg.at[j]),
                pltpu.async_copy(pr_h.at[irv.at[j]], bB.at[j], semg.at[j]),
            ))
        dw = []
        for j in range(NBUF_G):
            off = base + (g * NBUF_G + j) * CHG
            dg[j][0].wait()
            dg[j][1].wait()

            def addrow(r, c2, j=j):
                for k in range(GW // 16):
                    sl = pl.ds(k * 16, 16)
                    bA[j, r, sl] = bA[j, r, sl] + bB[j, r, sl]
                return c2

            lax.fori_loop(0, CHG, addrow, 0, unroll=4)
            dw.append(pltpu.async_copy(bA.at[j], out_h.at[pl.ds(off, CHG)],
                                       semw.at[j]))
        for j in range(NBUF_G):
            dw[j].wait()
        return carry

    lax.fori_loop(0, nch // NBUF_G, group, 0)


@functools.cache
def _sc_gather_fn(ne):
    body = functools.partial(_sc_gather_body, epw=ne // NW)
    return pl.kernel(
        body,
        out_type=jax.ShapeDtypeStruct((ne, GW), jnp.float32),
        mesh=plsc.VectorSubcoreMesh(core_axis_name="c", subcore_axis_name="s",
                                    num_cores=NC, num_subcores=NS),
        scratch_types=[
            pltpu.VMEM((NBUF_G, CHG), jnp.int32),
            pltpu.VMEM((NBUF_G, CHG), jnp.int32),
            pltpu.VMEM((NBUF_G, CHG, GW), jnp.float32),
            pltpu.VMEM((NBUF_G, CHG, GW), jnp.float32),
            pltpu.SemaphoreType.DMA((NBUF_G,)),
            pltpu.SemaphoreType.DMA((NBUF_G,)),
            pltpu.SemaphoreType.DMA((NBUF_G,)),
        ],
    )


def _sc_gather(ps, pr, sidx, ridx):
    return _sc_gather_fn(sidx.shape[0])(ps, pr, sidx, ridx)


def _sc_scatter_body(ed_h, ri_h, out_h, idxv, ebuf, acc, semi, seme, sems,
                     *, epw):
    cid = lax.axis_index("c")
    sid = lax.axis_index("s")
    wid = sid * NC + cid
    base = wid * epw
    nch = epw // CH

    def zrow(r, c2):
        for k in range(SW // 16):
            ebuf[0, r, pl.ds(k * 16, 16)] = jnp.zeros((16,), jnp.float32)
        return c2

    lax.fori_loop(0, CH, zrow, 0, unroll=2)
    rows_per_tile = N_ // NS
    pltpu.sync_copy(ebuf.at[0], acc.at[pl.ds(sid * rows_per_tile, CH)])
    plsc.subcore_barrier()

    def group(g, carry):
        dl = []
        for j in range(NBUF_S):
            off = base + (g * NBUF_S + j) * CH
            dl.append((
                pltpu.async_copy(ri_h.at[pl.ds(off, CH)], idxv.at[j], semi.at[j]),
                pltpu.async_copy(ed_h.at[pl.ds(off, CH)], ebuf.at[j], seme.at[j]),
            ))
        ds_ = []
        for j in range(NBUF_S):
            dl[j][0].wait()
            dl[j][1].wait()
            ds_.append(pltpu.async_copy(ebuf.at[j], acc.at[idxv.at[j]],
                                        sems.at[j], add=True))
        for j in range(NBUF_S):
            ds_[j].wait()
        return carry

    lax.fori_loop(0, nch // NBUF_S, group, 0)
    plsc.subcore_barrier()
    pltpu.sync_copy(acc.at[pl.ds(sid * rows_per_tile, rows_per_tile)],
                    out_h.at[pl.ds(cid * N_ + sid * rows_per_tile, rows_per_tile)])


@functools.cache
def _sc_scatter_fn(ne):
    body = functools.partial(_sc_scatter_body, epw=ne // NW)
    return pl.kernel(
        body,
        out_type=jax.ShapeDtypeStruct((NC * N_, SW), jnp.float32),
        mesh=plsc.VectorSubcoreMesh(core_axis_name="c", subcore_axis_name="s",
                                    num_cores=NC, num_subcores=NS),
        scratch_types=[
            pltpu.VMEM((NBUF_S, CH), jnp.int32),
            pltpu.VMEM((NBUF_S, CH, SW), jnp.float32),
            pltpu.VMEM_SHARED((N_, SW), jnp.float32),
            pltpu.SemaphoreType.DMA((NBUF_S,)),
            pltpu.SemaphoreType.DMA((NBUF_S,)),
            pltpu.SemaphoreType.DMA((NBUF_S,)),
        ],
    )


def _sc_scatter(edgesT, ridx):
    return _sc_scatter_fn(ridx.shape[0])(edgesT, ridx)


_BE = 2048
_BN = 512


def _edge_tc_body(gs_ref, ew_ref, w1e_ref, b1_ref, w2_ref, b2_ref, outT_ref):
    f32 = jnp.float32
    for b in range(B_):
        gs = gs_ref[:, b * EH:(b + 1) * EH]
        ep = jnp.dot(ew_ref[:, b * EO:(b + 1) * EO], w1e_ref[...],
                     preferred_element_type=f32)
        h = jnp.maximum(gs + ep + b1_ref[...], 0.0)
        en = jnp.dot(h, w2_ref[...], preferred_element_type=f32) + b2_ref[...]
        outT_ref[:, b * EO:(b + 1) * EO] = en


def _edge_tc(gsum, edgesT, w1e, b1, w2, b2):
    ne = gsum.shape[0]
    grid = (ne // _BE,)
    return pl.pallas_call(
        _edge_tc_body,
        grid=grid,
        in_specs=[
            pl.BlockSpec((_BE, GW), lambda i: (i, 0)),
            pl.BlockSpec((_BE, SW), lambda i: (i, 0)),
            pl.BlockSpec((EO, EH), lambda i: (0, 0)),
            pl.BlockSpec((1, EH), lambda i: (0, 0)),
            pl.BlockSpec((EH, EO), lambda i: (0, 0)),
            pl.BlockSpec((1, EO), lambda i: (0, 0)),
        ],
        out_specs=pl.BlockSpec((_BE, SW), lambda i: (i, 0)),
        out_shape=jax.ShapeDtypeStruct((ne, SW), jnp.float32),
    )(gsum, edgesT, w1e, b1, w2, b2)


def _proj_tc_body(x_ref, ws_ref, wr_ref, ps_ref, pr_ref):
    for b in range(B_):
        x = x_ref[b]
        ps_ref[:, b * EH:(b + 1) * EH] = jnp.dot(
            x, ws_ref[...], preferred_element_type=jnp.float32)
        pr_ref[:, b * EH:(b + 1) * EH] = jnp.dot(
            x, wr_ref[...], preferred_element_type=jnp.float32)


def _proj_tc(x, ws, wr):
    grid = (N_ // _BN,)
    return pl.pallas_call(
        _proj_tc_body,
        grid=grid,
        in_specs=[
            pl.BlockSpec((B_, _BN, NO), lambda i: (0, i, 0)),
            pl.BlockSpec((NO, EH), lambda i: (0, 0)),
            pl.BlockSpec((NO, EH), lambda i: (0, 0)),
        ],
        out_specs=[
            pl.BlockSpec((_BN, GW), lambda i: (i, 0)),
            pl.BlockSpec((_BN, GW), lambda i: (i, 0)),
        ],
        out_shape=[
            jax.ShapeDtypeStruct((N_, GW), jnp.float32),
            jax.ShapeDtypeStruct((N_, GW), jnp.float32),
        ],
    )(x, ws, wr)


def _node0_tc_body(nd_ref, pp_ref, pq_ref, w1n_ref, w1a_ref, b1_ref,
                   w2_ref, b2_ref, ws_ref, wr_ref, nd1_ref, ps_ref, pr_ref):
    f32 = jnp.float32
    for b in range(B_):
        sl = slice(b * EO, (b + 1) * EO)
        agg = (pp_ref[0, :, sl] + pp_ref[1, :, sl]
               + pq_ref[0, :, sl] + pq_ref[1, :, sl]) * (1.0 / N_)
        h = jnp.dot(nd_ref[b], w1n_ref[...], preferred_element_type=f32)
        h = h + jnp.dot(agg, w1a_ref[...], preferred_element_type=f32)
        h = jnp.maximum(h + b1_ref[...], 0.0)
        nd1 = jnp.dot(h, w2_ref[...], preferred_element_type=f32) + b2_ref[...]
        nd1_ref[b] = nd1
        ps_ref[:, b * EH:(b + 1) * EH] = jnp.dot(
            nd1, ws_ref[...], preferred_element_type=f32)
        pr_ref[:, b * EH:(b + 1) * EH] = jnp.dot(
            nd1, wr_ref[...], preferred_element_type=f32)


def _node0_tc(nodes, parts, parts2, w1n, w1a, b1, w2, b2, ws_next, wr_next):
    grid = (N_ // _BN,)
    return pl.pallas_call(
        _node0_tc_body,
        grid=grid,
        in_specs=[
            pl.BlockSpec((B_, _BN, NO), lambda i: (0, i, 0)),
            pl.BlockSpec((NC, _BN, SW), lambda i: (0, i, 0)),
            pl.BlockSpec((NC, _BN, SW), lambda i: (0, i, 0)),
            pl.BlockSpec((NO, NH), lambda i: (0, 0)),
            pl.BlockSpec((EO, NH), lambda i: (0, 0)),
            pl.BlockSpec((1, NH), lambda i: (0, 0)),
            pl.BlockSpec((NH, NO), lambda i: (0, 0)),
            pl.BlockSpec((1, NO), lambda i: (0, 0)),
            pl.BlockSpec((NO, EH), lambda i: (0, 0)),
            pl.BlockSpec((NO, EH), lambda i: (0, 0)),
        ],
        out_specs=[
            pl.BlockSpec((B_, _BN, NO), lambda i: (0, i, 0)),
            pl.BlockSpec((_BN, GW), lambda i: (i, 0)),
            pl.BlockSpec((_BN, GW), lambda i: (i, 0)),
        ],
        out_shape=[
            jax.ShapeDtypeStruct((B_, N_, NO), jnp.float32),
            jax.ShapeDtypeStruct((N_, GW), jnp.float32),
            jax.ShapeDtypeStruct((N_, GW), jnp.float32),
        ],
    )(nodes, parts, parts2, w1n, w1a, b1, w2, b2, ws_next, wr_next)


def _tail_tc_body(nd_ref, pp_ref, pq_ref, st_ref, cur_ref, eps_ref,
                  w1n_ref, w1a_ref, b1_ref, w2_ref, b2_ref,
                  wm1_ref, bm1_ref, wm2_ref, bm2_ref,
                  ws1_ref, bs1_ref, ws2_ref, bs2_ref,
                  wih_ref, whh_ref, bih_ref, bhh_ref,
                  wps_ref, wpr_ref,
                  mean_ref, lsd_ref, smp_ref, stn_ref, ps_ref, pr_ref):
    f32 = jnp.float32
    for b in range(B_):
        sl = slice(b * EO, (b + 1) * EO)
        agg = (pp_ref[0, :, sl] + pp_ref[1, :, sl]
               + pq_ref[0, :, sl] + pq_ref[1, :, sl]) * (1.0 / N_)
        h = jnp.dot(nd_ref[b], w1n_ref[...], preferred_element_type=f32)
        h = h + jnp.dot(agg, w1a_ref[...], preferred_element_type=f32)
        h = jnp.maximum(h + b1_ref[...], 0.0)
        gnn = jnp.dot(h, w2_ref[...], preferred_element_type=f32) + b2_ref[...]

        hm = jnp.maximum(jnp.dot(gnn, wm1_ref[...], preferred_element_type=f32)
                         + bm1_ref[...], 0.0)
        mean = (jnp.dot(hm, wm2_ref[...], preferred_element_type=f32)
                + bm2_ref[...] + cur_ref[b])
        hs = jnp.maximum(jnp.dot(gnn, ws1_ref[...], preferred_element_type=f32)
                         + bs1_ref[...], 0.0)
        lsd = jnp.clip(jnp.dot(hs, ws2_ref[...], preferred_element_type=f32)
                       + bs2_ref[...], -10.0, 10.0)
        mean_ref[b] = mean
        lsd_ref[b] = lsd
        smp_ref[b] = mean + jnp.exp(lsd) * eps_ref[b]

        st = st_ref[b]
        gi = jnp.dot(mean, wih_ref[...], preferred_element_type=f32) + bih_ref[...]
        gh = jnp.dot(st, whh_ref[...], preferred_element_type=f32) + bhh_ref[...]
        ir, iz, inn = gi[:, :RNN], gi[:, RNN:2 * RNN], gi[:, 2 * RNN:]
        hr, hz, hn = gh[:, :RNN], gh[:, RNN:2 * RNN], gh[:, 2 * RNN:]
        rr = jax.nn.sigmoid(ir + hr)
        z = jax.nn.sigmoid(iz + hz)
        nn_ = jnp.tanh(inn + rr * hn)
        stn = (1.0 - z) * nn_ + z * st
        stn_ref[b] = stn
        ps_ref[:, b * EH:(b + 1) * EH] = jnp.dot(
            stn, wps_ref[...], preferred_element_type=f32)
        pr_ref[:, b * EH:(b + 1) * EH] = jnp.dot(
            stn, wpr_ref[...], preferred_element_type=f32)


def _tail_tc(nodes, parts, parts2, state, cur, eps, P):
    grid = (N_ // _BN,)
    brow = lambda n2: pl.BlockSpec((B_, _BN, n2), lambda i: (0, i, 0))
    trow = lambda n2: pl.BlockSpec((_BN, n2), lambda i: (i, 0))
    full = lambda a, b: pl.BlockSpec((a, b), lambda i: (0, 0))
    return pl.pallas_call(
        _tail_tc_body,
        grid=grid,
        in_specs=[
            brow(NO),
            pl.BlockSpec((NC, _BN, SW), lambda i: (0, i, 0)),
            pl.BlockSpec((NC, _BN, SW), lambda i: (0, i, 0)),
            brow(RNN), brow(OUT), brow(OUT),
            full(NO, NH), full(EO, NH), full(1, NH), full(NH, NO), full(1, NO),
            full(NO, NO // 2), full(1, NO // 2), full(NO // 2, OUT), full(1, OUT),
            full(NO, NO // 2), full(1, NO // 2), full(NO // 2, OUT), full(1, OUT),
            full(DIN, 3 * RNN), full(RNN, 3 * RNN), full(1, 3 * RNN), full(1, 3 * RNN),
            full(NO, EH), full(NO, EH),
        ],
        out_specs=[brow(OUT), brow(OUT), brow(OUT), brow(RNN),
                   trow(GW), trow(GW)],
        out_shape=[
            jax.ShapeDtypeStruct((B_, N_, OUT), jnp.float32),
            jax.ShapeDtypeStruct((B_, N_, OUT), jnp.float32),
            jax.ShapeDtypeStruct((B_, N_, OUT), jnp.float32),
            jax.ShapeDtypeStruct((B_, N_, RNN), jnp.float32),
            jax.ShapeDtypeStruct((N_, GW), jnp.float32),
            jax.ShapeDtypeStruct((N_, GW), jnp.float32),
        ],
    )(nodes, parts, parts2, state, cur, eps, *P)


def _ews_tc_body(ed_ref, *rest):
    out_ref = rest[-1]
    x = ed_ref[...]
    x = x.reshape(_BE, B_, EO).transpose(1, 0, 2).reshape(B_, _BE * EO)
    out_ref[0] = x


def _ews_into(buf, ed, step, half):
    nb = E2 // _BE
    out_spec = pl.BlockSpec((1, B_, _BE * EO),
                            lambda j: (step, 0, half * nb + j))
    out_shape = jax.ShapeDtypeStruct((T_, B_, E_ * EO), jnp.float32)
    ed_spec = pl.BlockSpec((_BE, SW), lambda j: (j, 0))
    if buf is None:
        return pl.pallas_call(
            _ews_tc_body, grid=(nb,), in_specs=[ed_spec],
            out_specs=out_spec, out_shape=out_shape)(ed)
    return pl.pallas_call(
        _ews_tc_body, grid=(nb,),
        in_specs=[ed_spec, pl.BlockSpec(memory_space=pl.ANY)],
        out_specs=out_spec, out_shape=out_shape,
        input_output_aliases={1: 0})(ed, buf)


def kernel(input, state, edge_weight, rec_idx, send_idx,
           W_ih, W_hh, b_ih, b_hh,
           We1_0, be1_0, We2_0, be2_0, Wn1_0, bn1_0, Wn2_0, bn2_0,
           We1_1, be1_1, We2_1, be2_1, Wn1_1, bn1_1, Wn2_1, bn2_1,
           Wm1, bm1, Wm2, bm2, Ws1, bs1, Ws2, bs2):
    f32 = jnp.float32
    st = state
    edT = edge_weight.transpose(1, 0, 2).reshape(E_, SW)
    ed0, ed1 = edT[:E2], edT[E2:]
    cur = input[0]
    sidx = send_idx.astype(jnp.int32)
    ridx = rec_idx.astype(jnp.int32)
    si0, si1 = sidx[:E2], sidx[E2:]
    ri0, ri1 = ridx[:E2], ridx[E2:]

    r1 = lambda v: v.reshape(1, -1)
    Wp = [
        dict(w1s=We1_0[:NO], w1r=We1_0[NO:2 * NO], w1e=We1_0[2 * NO:],
             b1=r1(be1_0), w2=We2_0, b2=r1(be2_0),
             wn1n=Wn1_0[:NO], wn1a=Wn1_0[NO:], bn1=r1(bn1_0),
             wn2=Wn2_0, bn2=r1(bn2_0)),
        dict(w1s=We1_1[:NO], w1r=We1_1[NO:2 * NO], w1e=We1_1[2 * NO:],
             b1=r1(be1_1), w2=We2_1, b2=r1(be2_1),
             wn1n=Wn1_1[:NO], wn1a=Wn1_1[NO:], bn1=r1(bn1_1),
             wn2=Wn2_1, bn2=r1(bn2_1)),
    ]
    tailP = (Wp[1]['wn1n'], Wp[1]['wn1a'], Wp[1]['bn1'], Wp[1]['wn2'],
             Wp[1]['bn2'],
             Wm1, r1(bm1), Wm2, r1(bm2), Ws1, r1(bs1), Ws2, r1(bs2),
             W_ih, W_hh, r1(b_ih), r1(b_hh),
             Wp[0]['w1s'], Wp[0]['w1r'])

    nkey = jax.random.key(42)
    eps_all = [jax.random.normal(jax.random.fold_in(nkey, i), (B_, N_, OUT),
                                 f32) for i in range(T_)]

    ps0, pr0 = _proj_tc(st, Wp[0]['w1s'], Wp[0]['w1r'])

    means, lsds, smps = [], [], []
    ews_buf = None
    for i in range(T_):
        ews_buf = _ews_into(ews_buf, ed0, i, 0)
        ews_buf = _ews_into(ews_buf, ed1, i, 1)
        for p in range(2):
            g0 = _sc_gather(ps0, pr0, si0, ri0)
            g1 = _sc_gather(ps0, pr0, si1, ri1)
            ed0 = _edge_tc(g0, ed0, Wp[p]['w1e'], Wp[p]['b1'],
                           Wp[p]['w2'], Wp[p]['b2'])
            ed1 = _edge_tc(g1, ed1, Wp[p]['w1e'], Wp[p]['b1'],
                           Wp[p]['w2'], Wp[p]['b2'])
            pa = _sc_scatter(ed0, ri0).reshape(NC, N_, SW)
            pb = _sc_scatter(ed1, ri1).reshape(NC, N_, SW)
            if p == 0:
                nd1, ps0, pr0 = _node0_tc(st, pa, pb, Wp[0]['wn1n'],
                                          Wp[0]['wn1a'], Wp[0]['bn1'],
                                          Wp[0]['wn2'], Wp[0]['bn2'],
                                          Wp[1]['w1s'], Wp[1]['w1r'])
            else:
                mean, lsd, smp, st, ps0, pr0 = _tail_tc(nd1, pa, pb, st, cur,
                                                        eps_all[i], tailP)
        cur = mean
        means.append(mean.reshape(1, B_, N_ * OUT))
        lsds.append(lsd.reshape(1, B_, N_ * OUT))
        smps.append(smp.reshape(1, B_, N_ * OUT))

    return (jnp.concatenate(means, 0), jnp.concatenate(lsds, 0),
            jnp.concatenate(smps, 0), st, ews_buf)


# speedup vs baseline: 1.8031x; 1.1132x over previous
import functools

import jax
import jax.numpy as jnp
from jax import lax
from jax.experimental import pallas as pl
from jax.experimental.pallas import tpu as pltpu
from jax.experimental.pallas import tpu_sc as plsc

T_, B_, N_, E_ = 8, 4, 2048, 32768
RNN, NH, NO, EH, EO, OUT, DIN = 64, 64, 64, 64, 32, 6, 6

NC, NS = 2, 16
NW = NC * NS
GW = B_ * EH
SW = B_ * EO
E2 = E_ // 2
CH = 128
CHG = 64
NBUF_G = 2
NBUF_S = 4


def _sc_gather_body(ps_h, pr_h, si_h, ri_h, out_h, isv, irv, bA, bB, bO,
                    semi, semg, semw, *, epw):
    wid = lax.axis_index("s") * NC + lax.axis_index("c")
    base = wid * epw
    nch = epw // CHG

    di = (pltpu.async_copy(si_h.at[pl.ds(base, epw)], isv, semi),
          pltpu.async_copy(ri_h.at[pl.ds(base, epw)], irv, semi))
    di[0].wait()
    di[1].wait()

    def fire(ci):
        s = ci % NBUF_G
        sl = pl.ds(ci * CHG, CHG)
        return (pltpu.async_copy(ps_h.at[isv.at[sl]], bA.at[s], semg.at[s]),
                pltpu.async_copy(pr_h.at[irv.at[sl]], bB.at[s], semg.at[s]))

    gd = [None] * nch
    wd = [None] * nch
    for ci in range(min(NBUF_G, nch)):
        gd[ci] = fire(ci)
    for ci in range(nch):
        s = ci % NBUF_G
        gd[ci][0].wait()
        gd[ci][1].wait()
        if ci >= NBUF_G:
            wd[ci - NBUF_G].wait()

        def addrow(r, c2, s=s):
            for k in range(GW // 16):
                sl = pl.ds(k * 16, 16)
                bO[s, r, sl] = bA[s, r, sl] + bB[s, r, sl]
            return c2

        lax.fori_loop(0, CHG, addrow, 0, unroll=4)
        nx = ci + NBUF_G
        if nx < nch:
            gd[nx] = fire(nx)
        wd[ci] = pltpu.async_copy(bO.at[s],
                                  out_h.at[pl.ds(base + ci * CHG, CHG)],
                                  semw.at[s])
    for ci in range(max(0, nch - NBUF_G), nch):
        wd[ci].wait()


@functools.cache
def _sc_gather_fn(ne):
    epw = ne // NW
    body = functools.partial(_sc_gather_body, epw=epw)
    return pl.kernel(
        body,
        out_type=jax.ShapeDtypeStruct((ne, GW), jnp.float32),
        mesh=plsc.VectorSubcoreMesh(core_axis_name="c", subcore_axis_name="s",
                                    num_cores=NC, num_subcores=NS),
        scratch_types=[
            pltpu.VMEM((epw,), jnp.int32),
            pltpu.VMEM((epw,), jnp.int32),
            pltpu.VMEM((NBUF_G, CHG, GW), jnp.float32),
            pltpu.VMEM((NBUF_G, CHG, GW), jnp.float32),
            pltpu.VMEM((NBUF_G, CHG, GW), jnp.float32),
            pltpu.SemaphoreType.DMA,
            pltpu.SemaphoreType.DMA((NBUF_G,)),
            pltpu.SemaphoreType.DMA((NBUF_G,)),
        ],
    )


def _sc_gather(ps, pr, sidx, ridx):
    return _sc_gather_fn(sidx.shape[0])(ps, pr, sidx, ridx)


def _sc_scatter_body(ed_h, ri_h, out_h, idxv, ebuf, acc, semi, seme, sems,
                     *, epw):
    cid = lax.axis_index("c")
    sid = lax.axis_index("s")
    wid = sid * NC + cid
    base = wid * epw
    nch = epw // CH

    def zrow(r, c2):
        for k in range(SW // 16):
            ebuf[0, r, pl.ds(k * 16, 16)] = jnp.zeros((16,), jnp.float32)
        return c2

    lax.fori_loop(0, CH, zrow, 0, unroll=2)
    rows_per_tile = N_ // NS
    pltpu.sync_copy(ebuf.at[0], acc.at[pl.ds(sid * rows_per_tile, CH)])
    plsc.subcore_barrier()

    def group(g, carry):
        dl = []
        for j in range(NBUF_S):
            off = base + (g * NBUF_S + j) * CH
            dl.append((
                pltpu.async_copy(ri_h.at[pl.ds(off, CH)], idxv.at[j], semi.at[j]),
                pltpu.async_copy(ed_h.at[pl.ds(off, CH)], ebuf.at[j], seme.at[j]),
            ))
        ds_ = []
        for j in range(NBUF_S):
            dl[j][0].wait()
            dl[j][1].wait()
            ds_.append(pltpu.async_copy(ebuf.at[j], acc.at[idxv.at[j]],
                                        sems.at[j], add=True))
        for j in range(NBUF_S):
            ds_[j].wait()
        return carry

    lax.fori_loop(0, nch // NBUF_S, group, 0)
    plsc.subcore_barrier()
    pltpu.sync_copy(acc.at[pl.ds(sid * rows_per_tile, rows_per_tile)],
                    out_h.at[pl.ds(cid * N_ + sid * rows_per_tile, rows_per_tile)])


@functools.cache
def _sc_scatter_fn(ne):
    body = functools.partial(_sc_scatter_body, epw=ne // NW)
    return pl.kernel(
        body,
        out_type=jax.ShapeDtypeStruct((NC * N_, SW), jnp.float32),
        mesh=plsc.VectorSubcoreMesh(core_axis_name="c", subcore_axis_name="s",
                                    num_cores=NC, num_subcores=NS),
        scratch_types=[
            pltpu.VMEM((NBUF_S, CH), jnp.int32),
            pltpu.VMEM((NBUF_S, CH, SW), jnp.float32),
            pltpu.VMEM_SHARED((N_, SW), jnp.float32),
            pltpu.SemaphoreType.DMA((NBUF_S,)),
            pltpu.SemaphoreType.DMA((NBUF_S,)),
            pltpu.SemaphoreType.DMA((NBUF_S,)),
        ],
    )


def _sc_scatter(edgesT, ridx):
    return _sc_scatter_fn(ridx.shape[0])(edgesT, ridx)


_BE = 2048
_BN = 512


def _edge_tc_body(gs_ref, ew_ref, w1e_ref, b1_ref, w2_ref, b2_ref, outT_ref):
    f32 = jnp.float32
    for b in range(B_):
        gs = gs_ref[:, b * EH:(b + 1) * EH]
        ep = jnp.dot(ew_ref[:, b * EO:(b + 1) * EO], w1e_ref[...],
                     preferred_element_type=f32)
        h = jnp.maximum(gs + ep + b1_ref[...], 0.0)
        en = jnp.dot(h, w2_ref[...], preferred_element_type=f32) + b2_ref[...]
        outT_ref[:, b * EO:(b + 1) * EO] = en


def _edge_tc(gsum, edgesT, w1e, b1, w2, b2):
    ne = gsum.shape[0]
    grid = (ne // _BE,)
    return pl.pallas_call(
        _edge_tc_body,
        grid=grid,
        in_specs=[
            pl.BlockSpec((_BE, GW), lambda i: (i, 0)),
            pl.BlockSpec((_BE, SW), lambda i: (i, 0)),
            pl.BlockSpec((EO, EH), lambda i: (0, 0)),
            pl.BlockSpec((1, EH), lambda i: (0, 0)),
            pl.BlockSpec((EH, EO), lambda i: (0, 0)),
            pl.BlockSpec((1, EO), lambda i: (0, 0)),
        ],
        out_specs=pl.BlockSpec((_BE, SW), lambda i: (i, 0)),
        out_shape=jax.ShapeDtypeStruct((ne, SW), jnp.float32),
    )(gsum, edgesT, w1e, b1, w2, b2)


def _proj_tc_body(x_ref, ws_ref, wr_ref, ps_ref, pr_ref):
    for b in range(B_):
        x = x_ref[b]
        ps_ref[:, b * EH:(b + 1) * EH] = jnp.dot(
            x, ws_ref[...], preferred_element_type=jnp.float32)
        pr_ref[:, b * EH:(b + 1) * EH] = jnp.dot(
            x, wr_ref[...], preferred_element_type=jnp.float32)


def _proj_tc(x, ws, wr):
    grid = (N_ // _BN,)
    return pl.pallas_call(
        _proj_tc_body,
        grid=grid,
        in_specs=[
            pl.BlockSpec((B_, _BN, NO), lambda i: (0, i, 0)),
            pl.BlockSpec((NO, EH), lambda i: (0, 0)),
            pl.BlockSpec((NO, EH), lambda i: (0, 0)),
        ],
        out_specs=[
            pl.BlockSpec((_BN, GW), lambda i: (i, 0)),
            pl.BlockSpec((_BN, GW), lambda i: (i, 0)),
        ],
        out_shape=[
            jax.ShapeDtypeStruct((N_, GW), jnp.float32),
            jax.ShapeDtypeStruct((N_, GW), jnp.float32),
        ],
    )(x, ws, wr)


def _node0_tc_body(nd_ref, pp_ref, pq_ref, w1n_ref, w1a_ref, b1_ref,
                   w2_ref, b2_ref, ws_ref, wr_ref, nd1_ref, ps_ref, pr_ref):
    f32 = jnp.float32
    for b in range(B_):
        sl = slice(b * EO, (b + 1) * EO)
        agg = (pp_ref[0, :, sl] + pp_ref[1, :, sl]
               + pq_ref[0, :, sl] + pq_ref[1, :, sl]) * (1.0 / N_)
        h = jnp.dot(nd_ref[b], w1n_ref[...], preferred_element_type=f32)
        h = h + jnp.dot(agg, w1a_ref[...], preferred_element_type=f32)
        h = jnp.maximum(h + b1_ref[...], 0.0)
        nd1 = jnp.dot(h, w2_ref[...], preferred_element_type=f32) + b2_ref[...]
        nd1_ref[b] = nd1
        ps_ref[:, b * EH:(b + 1) * EH] = jnp.dot(
            nd1, ws_ref[...], preferred_element_type=f32)
        pr_ref[:, b * EH:(b + 1) * EH] = jnp.dot(
            nd1, wr_ref[...], preferred_element_type=f32)


def _node0_tc(nodes, parts, parts2, w1n, w1a, b1, w2, b2, ws_next, wr_next):
    grid = (N_ // _BN,)
    return pl.pallas_call(
        _node0_tc_body,
        grid=grid,
        in_specs=[
            pl.BlockSpec((B_, _BN, NO), lambda i: (0, i, 0)),
            pl.BlockSpec((NC, _BN, SW), lambda i: (0, i, 0)),
            pl.BlockSpec((NC, _BN, SW), lambda i: (0, i, 0)),
            pl.BlockSpec((NO, NH), lambda i: (0, 0)),
            pl.BlockSpec((EO, NH), lambda i: (0, 0)),
            pl.BlockSpec((1, NH), lambda i: (0, 0)),
            pl.BlockSpec((NH, NO), lambda i: (0, 0)),
            pl.BlockSpec((1, NO), lambda i: (0, 0)),
            pl.BlockSpec((NO, EH), lambda i: (0, 0)),
            pl.BlockSpec((NO, EH), lambda i: (0, 0)),
        ],
        out_specs=[
            pl.BlockSpec((B_, _BN, NO), lambda i: (0, i, 0)),
            pl.BlockSpec((_BN, GW), lambda i: (i, 0)),
            pl.BlockSpec((_BN, GW), lambda i: (i, 0)),
        ],
        out_shape=[
            jax.ShapeDtypeStruct((B_, N_, NO), jnp.float32),
            jax.ShapeDtypeStruct((N_, GW), jnp.float32),
            jax.ShapeDtypeStruct((N_, GW), jnp.float32),
        ],
    )(nodes, parts, parts2, w1n, w1a, b1, w2, b2, ws_next, wr_next)


def _tail_tc_body(nd_ref, pp_ref, pq_ref, st_ref, cur_ref, eps_ref,
                  w1n_ref, w1a_ref, b1_ref, w2_ref, b2_ref,
                  wm1_ref, bm1_ref, wm2_ref, bm2_ref,
                  ws1_ref, bs1_ref, ws2_ref, bs2_ref,
                  wih_ref, whh_ref, bih_ref, bhh_ref,
                  wps_ref, wpr_ref,
                  mean_ref, lsd_ref, smp_ref, stn_ref, ps_ref, pr_ref):
    f32 = jnp.float32
    for b in range(B_):
        sl = slice(b * EO, (b + 1) * EO)
        agg = (pp_ref[0, :, sl] + pp_ref[1, :, sl]
               + pq_ref[0, :, sl] + pq_ref[1, :, sl]) * (1.0 / N_)
        h = jnp.dot(nd_ref[b], w1n_ref[...], preferred_element_type=f32)
        h = h + jnp.dot(agg, w1a_ref[...], preferred_element_type=f32)
        h = jnp.maximum(h + b1_ref[...], 0.0)
        gnn = jnp.dot(h, w2_ref[...], preferred_element_type=f32) + b2_ref[...]

        hm = jnp.maximum(jnp.dot(gnn, wm1_ref[...], preferred_element_type=f32)
                         + bm1_ref[...], 0.0)
        mean = (jnp.dot(hm, wm2_ref[...], preferred_element_type=f32)
                + bm2_ref[...] + cur_ref[b])
        hs = jnp.maximum(jnp.dot(gnn, ws1_ref[...], preferred_element_type=f32)
                         + bs1_ref[...], 0.0)
        lsd = jnp.clip(jnp.dot(hs, ws2_ref[...], preferred_element_type=f32)
                       + bs2_ref[...], -10.0, 10.0)
        mean_ref[b] = mean
        lsd_ref[b] = lsd
        smp_ref[b] = mean + jnp.exp(lsd) * eps_ref[b]

        st = st_ref[b]
        gi = jnp.dot(mean, wih_ref[...], preferred_element_type=f32) + bih_ref[...]
        gh = jnp.dot(st, whh_ref[...], preferred_element_type=f32) + bhh_ref[...]
        ir, iz, inn = gi[:, :RNN], gi[:, RNN:2 * RNN], gi[:, 2 * RNN:]
        hr, hz, hn = gh[:, :RNN], gh[:, RNN:2 * RNN], gh[:, 2 * RNN:]
        rr = jax.nn.sigmoid(ir + hr)
        z = jax.nn.sigmoid(iz + hz)
        nn_ = jnp.tanh(inn + rr * hn)
        stn = (1.0 - z) * nn_ + z * st
        stn_ref[b] = stn
        ps_ref[:, b * EH:(b + 1) * EH] = jnp.dot(
            stn, wps_ref[...], preferred_element_type=f32)
        pr_ref[:, b * EH:(b + 1) * EH] = jnp.dot(
            stn, wpr_ref[...], preferred_element_type=f32)


def _tail_tc(nodes, parts, parts2, state, cur, eps, P):
    grid = (N_ // _BN,)
    brow = lambda n2: pl.BlockSpec((B_, _BN, n2), lambda i: (0, i, 0))
    trow = lambda n2: pl.BlockSpec((_BN, n2), lambda i: (i, 0))
    full = lambda a, b: pl.BlockSpec((a, b), lambda i: (0, 0))
    return pl.pallas_call(
        _tail_tc_body,
        grid=grid,
        in_specs=[
            brow(NO),
            pl.BlockSpec((NC, _BN, SW), lambda i: (0, i, 0)),
            pl.BlockSpec((NC, _BN, SW), lambda i: (0, i, 0)),
            brow(RNN), brow(OUT), brow(OUT),
            full(NO, NH), full(EO, NH), full(1, NH), full(NH, NO), full(1, NO),
            full(NO, NO // 2), full(1, NO // 2), full(NO // 2, OUT), full(1, OUT),
            full(NO, NO // 2), full(1, NO // 2), full(NO // 2, OUT), full(1, OUT),
            full(DIN, 3 * RNN), full(RNN, 3 * RNN), full(1, 3 * RNN), full(1, 3 * RNN),
            full(NO, EH), full(NO, EH),
        ],
        out_specs=[brow(OUT), brow(OUT), brow(OUT), brow(RNN),
                   trow(GW), trow(GW)],
        out_shape=[
            jax.ShapeDtypeStruct((B_, N_, OUT), jnp.float32),
            jax.ShapeDtypeStruct((B_, N_, OUT), jnp.float32),
            jax.ShapeDtypeStruct((B_, N_, OUT), jnp.float32),
            jax.ShapeDtypeStruct((B_, N_, RNN), jnp.float32),
            jax.ShapeDtypeStruct((N_, GW), jnp.float32),
            jax.ShapeDtypeStruct((N_, GW), jnp.float32),
        ],
    )(nodes, parts, parts2, state, cur, eps, *P)


def _ews_tc_body(ed_ref, *rest):
    out_ref = rest[-1]
    x = ed_ref[...]
    x = x.reshape(_BE, B_, EO).transpose(1, 0, 2).reshape(B_, _BE * EO)
    out_ref[0] = x


def _ews_into(buf, ed, step, half):
    nb = E2 // _BE
    out_spec = pl.BlockSpec((1, B_, _BE * EO),
                            lambda j: (step, 0, half * nb + j))
    out_shape = jax.ShapeDtypeStruct((T_, B_, E_ * EO), jnp.float32)
    ed_spec = pl.BlockSpec((_BE, SW), lambda j: (j, 0))
    if buf is None:
        return pl.pallas_call(
            _ews_tc_body, grid=(nb,), in_specs=[ed_spec],
            out_specs=out_spec, out_shape=out_shape)(ed)
    return pl.pallas_call(
        _ews_tc_body, grid=(nb,),
        in_specs=[ed_spec, pl.BlockSpec(memory_space=pl.ANY)],
        out_specs=out_spec, out_shape=out_shape,
        input_output_aliases={1: 0})(ed, buf)


def kernel(input, state, edge_weight, rec_idx, send_idx,
           W_ih, W_hh, b_ih, b_hh,
           We1_0, be1_0, We2_0, be2_0, Wn1_0, bn1_0, Wn2_0, bn2_0,
           We1_1, be1_1, We2_1, be2_1, Wn1_1, bn1_1, Wn2_1, bn2_1,
           Wm1, bm1, Wm2, bm2, Ws1, bs1, Ws2, bs2):
    f32 = jnp.float32
    st = state
    edT = edge_weight.transpose(1, 0, 2).reshape(E_, SW)
    ed0, ed1 = edT[:E2], edT[E2:]
    cur = input[0]
    sidx = send_idx.astype(jnp.int32)
    ridx = rec_idx.astype(jnp.int32)
    si0, si1 = sidx[:E2], sidx[E2:]
    ri0, ri1 = ridx[:E2], ridx[E2:]

    r1 = lambda v: v.reshape(1, -1)
    Wp = [
        dict(w1s=We1_0[:NO], w1r=We1_0[NO:2 * NO], w1e=We1_0[2 * NO:],
             b1=r1(be1_0), w2=We2_0, b2=r1(be2_0),
             wn1n=Wn1_0[:NO], wn1a=Wn1_0[NO:], bn1=r1(bn1_0),
             wn2=Wn2_0, bn2=r1(bn2_0)),
        dict(w1s=We1_1[:NO], w1r=We1_1[NO:2 * NO], w1e=We1_1[2 * NO:],
             b1=r1(be1_1), w2=We2_1, b2=r1(be2_1),
             wn1n=Wn1_1[:NO], wn1a=Wn1_1[NO:], bn1=r1(bn1_1),
             wn2=Wn2_1, bn2=r1(bn2_1)),
    ]
    tailP = (Wp[1]['wn1n'], Wp[1]['wn1a'], Wp[1]['bn1'], Wp[1]['wn2'],
             Wp[1]['bn2'],
             Wm1, r1(bm1), Wm2, r1(bm2), Ws1, r1(bs1), Ws2, r1(bs2),
             W_ih, W_hh, r1(b_ih), r1(b_hh),
             Wp[0]['w1s'], Wp[0]['w1r'])

    nkey = jax.random.key(42)
    eps_all = [jax.random.normal(jax.random.fold_in(nkey, i), (B_, N_, OUT),
                                 f32) for i in range(T_)]

    ps0, pr0 = _proj_tc(st, Wp[0]['w1s'], Wp[0]['w1r'])

    means, lsds, smps = [], [], []
    ews_buf = None
    for i in range(T_):
        ews_buf = _ews_into(ews_buf, ed0, i, 0)
        ews_buf = _ews_into(ews_buf, ed1, i, 1)
        for p in range(2):
            g0 = _sc_gather(ps0, pr0, si0, ri0)
            g1 = _sc_gather(ps0, pr0, si1, ri1)
            ed0 = _edge_tc(g0, ed0, Wp[p]['w1e'], Wp[p]['b1'],
                           Wp[p]['w2'], Wp[p]['b2'])
            ed1 = _edge_tc(g1, ed1, Wp[p]['w1e'], Wp[p]['b1'],
                           Wp[p]['w2'], Wp[p]['b2'])
            pa = _sc_scatter(ed0, ri0).reshape(NC, N_, SW)
            pb = _sc_scatter(ed1, ri1).reshape(NC, N_, SW)
            if p == 0:
                nd1, ps0, pr0 = _node0_tc(st, pa, pb, Wp[0]['wn1n'],
                                          Wp[0]['wn1a'], Wp[0]['bn1'],
                                          Wp[0]['wn2'], Wp[0]['bn2'],
                                          Wp[1]['w1s'], Wp[1]['w1r'])
            else:
                mean, lsd, smp, st, ps0, pr0 = _tail_tc(nd1, pa, pb, st, cur,
                                                        eps_all[i], tailP)
        cur = mean
        means.append(mean.reshape(1, B_, N_ * OUT))
        lsds.append(lsd.reshape(1, B_, N_ * OUT))
        smps.append(smp.reshape(1, B_, N_ * OUT))

    return (jnp.concatenate(means, 0), jnp.concatenate(lsds, 0),
            jnp.concatenate(smps, 0), st, ews_buf)
